# baseline (speedup 1.0000x reference)
import functools

import jax
import jax.numpy as jnp
from jax import lax
from jax.experimental import pallas as pl
from jax.experimental.pallas import tpu as pltpu
from jax.experimental.pallas import tpu_sc as plsc

N_EXP = 8
D = 1024
DFF = 512
N_TOK = 2048
N_PAIR = 2 * N_TOK
BT = 256
G = N_PAIR // BT + N_EXP
P_ROWS = G * BT
CHUNK = 64
N_CHUNK = N_PAIR // CHUNK
NEG = -3.0e38


def _argmax_lanes(v, iota_row):
    m = jnp.max(v, axis=-1, keepdims=True)
    return jnp.min(jnp.where(v == m, iota_row, N_EXP), axis=-1, keepdims=True), m


def _routing_body(x_ref, gw_ref, sim_ref, d_ref, w_ref, bmap_ref, bexp_ref,
                  oc_ref, bt_ref):
    x = x_ref[...]
    gw = gw_ref[...]
    logits = lax.dot_general(x, gw, (((1,), (1,)), ((), ())),
                             preferred_element_type=jnp.float32)
    m = jnp.max(logits, axis=-1, keepdims=True)
    e = jnp.exp(logits - m)
    probs = e / jnp.sum(e, axis=-1, keepdims=True)

    iota_row = lax.broadcasted_iota(jnp.int32, (N_TOK, N_EXP), 1)
    i1, v1 = _argmax_lanes(probs, iota_row)
    oh1 = (iota_row == i1)
    probs2 = jnp.where(oh1, NEG, probs)
    i2, v2 = _argmax_lanes(probs2, iota_row)
    oh2 = (iota_row == i2)
    denom = jnp.maximum(v1 + v2, 1e-12)
    w1 = v1 / denom
    w2 = v2 / denom

    mask_col = jnp.max(oh1.astype(jnp.float32), axis=0, keepdims=True)
    sim = sim_ref[...]
    iota_r8 = lax.broadcasted_iota(jnp.int32, (N_EXP, N_EXP), 1)
    iota_c8 = lax.broadcasted_iota(jnp.int32, (N_EXP, N_EXP), 0)
    eye = (iota_r8 == iota_c8)
    sim_masked = jnp.where(mask_col > 0.5, sim, NEG)
    best_sim = jnp.max(sim_masked, axis=-1, keepdims=True)
    best_j = jnp.min(jnp.where(sim_masked == best_sim, iota_r8, N_EXP),
                     axis=-1, keepdims=True)
    mask_row = jnp.max(jnp.where(eye, jnp.broadcast_to(mask_col, (N_EXP, N_EXP)),
                                 0.0), axis=-1, keepdims=True)
    reroute = (mask_row < 0.5) & (best_sim >= 0.5)
    ident = lax.broadcasted_iota(jnp.int32, (N_EXP, 1), 0)
    emap = jnp.where(reroute, best_j, ident)
    perm = (emap == iota_r8).astype(jnp.float32)

    pre = w1 * oh1.astype(jnp.float32) + w2 * oh2.astype(jnp.float32)
    rw = lax.dot_general(pre, perm, (((1,), (0,)), ((), ())),
                         preferred_element_type=jnp.float32)

    f1, u1 = _argmax_lanes(rw, iota_row)
    ohf1 = (iota_row == f1)
    rwm = jnp.where(ohf1, -1.0, rw)
    f2, u2 = _argmax_lanes(rwm, iota_row)
    ohf2 = (iota_row == f2)

    w_ref[:N_TOK, :] = jnp.broadcast_to(u1, (N_TOK, 16))
    w_ref[N_TOK:, :] = jnp.broadcast_to(u2, (N_TOK, 16))

    iota_pr = lax.broadcasted_iota(jnp.int32, (128, 128), 0)
    iota_pc = lax.broadcasted_iota(jnp.int32, (128, 128), 1)
    tri128 = (iota_pc <= iota_pr).astype(jnp.float32)

    n_blk = N_PAIR // 128
    for b in range(n_blk):
        if b < n_blk // 2:
            o_blk = ohf1[b * 128:(b + 1) * 128, :].astype(jnp.float32)
        else:
            o_blk = ohf2[(b - n_blk // 2) * 128:(b - n_blk // 2 + 1) * 128,
                         :].astype(jnp.float32)
        c = lax.dot_general(tri128, o_blk, (((1,), (0,)), ((), ())),
                            preferred_element_type=jnp.float32)
        oc_ref[b * 128:(b + 1) * 128, :] = c
        bt_ref[b:b + 1, :] = c[127:128, :]

    iota_br = lax.broadcasted_iota(jnp.int32, (n_blk, n_blk), 0)
    iota_bc = lax.broadcasted_iota(jnp.int32, (n_blk, n_blk), 1)
    triS = (iota_bc < iota_br).astype(jnp.float32)
    btm = bt_ref[...]
    carry = lax.dot_general(triS, btm, (((1,), (0,)), ((), ())),
                            preferred_element_type=jnp.float32)

    counts = jnp.sum(btm, axis=0, keepdims=True)
    nb = jnp.floor((counts + float(BT - 1)) * (1.0 / BT))
    iota_u8 = lax.broadcasted_iota(jnp.int32, (N_EXP, N_EXP), 0)
    u8 = (iota_u8 < iota_r8).astype(jnp.float32)
    excl = lax.dot_general(nb, u8, (((1,), (0,)), ((), ())),
                           preferred_element_type=jnp.float32)
    pad_base = excl * float(BT)
    total = jnp.sum(nb)
    cumnext = excl + nb

    for b in range(n_blk):
        if b < n_blk // 2:
            o_blk = ohf1[b * 128:(b + 1) * 128, :].astype(jnp.float32)
        else:
            o_blk = ohf2[(b - n_blk // 2) * 128:(b - n_blk // 2 + 1) * 128,
                         :].astype(jnp.float32)
        inc = oc_ref[b * 128:(b + 1) * 128, :] + carry[b:b + 1, :]
        rank = jnp.sum(inc * o_blk, axis=-1, keepdims=True) - 1.0
        pb = jnp.sum(pad_base * o_blk, axis=-1, keepdims=True)
        d_ref[b * 128:(b + 1) * 128, :] = (pb + rank).astype(jnp.int32)

    gcol = lax.broadcasted_iota(jnp.int32, (G, 1), 0).astype(jnp.float32)
    bmapf = jnp.minimum(gcol, total - 1.0)
    bexp = jnp.sum((jnp.broadcast_to(cumnext, (G, N_EXP)) <= bmapf)
                   .astype(jnp.int32), axis=-1, keepdims=True)
    bmap_ref[...] = bmapf.astype(jnp.int32)
    bexp_ref[...] = bexp


def _routing(x, gate_weight, sim):
    return pl.pallas_call(
        _routing_body,
        out_shape=(
            jax.ShapeDtypeStruct((N_PAIR, 1), jnp.int32),
            jax.ShapeDtypeStruct((N_PAIR, 16), jnp.float32),
            jax.ShapeDtypeStruct((G, 1), jnp.int32),
            jax.ShapeDtypeStruct((G, 1), jnp.int32),
        ),
        in_specs=[
            pl.BlockSpec((N_TOK, D), lambda: (0, 0)),
            pl.BlockSpec((N_EXP, D), lambda: (0, 0)),
            pl.BlockSpec((N_EXP, N_EXP), lambda: (0, 0)),
        ],
        out_specs=(
            pl.BlockSpec((N_PAIR, 1), lambda: (0, 0)),
            pl.BlockSpec((N_PAIR, 16), lambda: (0, 0)),
            pl.BlockSpec((G, 1), lambda: (0, 0)),
            pl.BlockSpec((G, 1), lambda: (0, 0)),
        ),
        scratch_shapes=[
            pltpu.VMEM((N_PAIR, N_EXP), jnp.float32),
            pltpu.VMEM((N_PAIR // 128, N_EXP), jnp.float32),
        ],
    )(x, gate_weight, sim)


_NC = 2
_NS = 16
_NW = _NC * _NS


def _sc_scatter_body(x_hbm, dmat_hbm, xs_hbm, idx_v, rows_v, sem):
    wid = lax.axis_index("s") * _NC + lax.axis_index("c")
    for half in range(2):
        j = wid + half * _NW
        t0 = wid * CHUNK
        pltpu.sync_copy(dmat_hbm.at[j], idx_v)
        pltpu.sync_copy(x_hbm.at[pl.ds(t0, CHUNK)], rows_v)
        pltpu.async_copy(rows_v, xs_hbm.at[idx_v], sem).wait()


def _sc_scatter(x, d_mat):
    mesh = plsc.VectorSubcoreMesh(core_axis_name="c", subcore_axis_name="s", num_cores=_NC, num_subcores=_NS)
    f = pl.kernel(
        _sc_scatter_body,
        out_type=jax.ShapeDtypeStruct((P_ROWS, D), jnp.float32),
        mesh=mesh,
        scratch_types=[
            pltpu.VMEM((CHUNK,), jnp.int32),
            pltpu.VMEM((CHUNK, D), jnp.float32),
            pltpu.SemaphoreType.DMA,
        ],
    )
    return f(x, d_mat)


def _gemm_body(bmap_ref, bexp_ref, xs_ref, gup_ref, down_ref, y_ref):
    g = pl.program_id(0)

    @pl.when(bmap_ref[g, 0] == g)
    def _():
        xs = xs_ref[...].astype(jnp.bfloat16)
        gu = lax.dot_general(xs, gup_ref[0], (((1,), (1,)), ((), ())),
                             preferred_element_type=jnp.float32)
        gate = gu[:, :DFF]
        up = gu[:, DFF:]
        h = gate * jax.nn.sigmoid(gate) * up
        y_ref[...] = lax.dot_general(h.astype(jnp.bfloat16), down_ref[0],
                                     (((1,), (1,)), ((), ())),
                                     preferred_element_type=jnp.float32)


def _ragged_gemm(bmap, bexp, xs, gate_up_proj, down_proj):
    grid_spec = pltpu.PrefetchScalarGridSpec(
        num_scalar_prefetch=2,
        grid=(G,),
        in_specs=[
            pl.BlockSpec((BT, D), lambda g, bm, be: (bm[g, 0], 0)),
            pl.BlockSpec((1, 2 * DFF, D), lambda g, bm, be: (be[g, 0], 0, 0)),
            pl.BlockSpec((1, D, DFF), lambda g, bm, be: (be[g, 0], 0, 0)),
        ],
        out_specs=pl.BlockSpec((BT, D), lambda g, bm, be: (bm[g, 0], 0)),
    )
    return pl.pallas_call(
        _gemm_body,
        grid_spec=grid_spec,
        out_shape=jax.ShapeDtypeStruct((P_ROWS, D), jnp.float32),
    )(bmap, bexp, xs, gate_up_proj, down_proj)


_TSUB = 16


def _sc_combine_body(y_hbm, dmat_hbm, ws_hbm, out_hbm,
                     i0_v, i1_v, w0s_v, w1s_v, a_v, b_v, o_v, sem):
    wid = lax.axis_index("s") * _NC + lax.axis_index("c")
    for s in range(CHUNK // _TSUB):
        col = s * _TSUB
        row0 = wid * CHUNK + col
        pltpu.sync_copy(dmat_hbm.at[wid, pl.ds(col, _TSUB)], i0_v)
        pltpu.sync_copy(dmat_hbm.at[wid + _NW, pl.ds(col, _TSUB)], i1_v)
        pltpu.sync_copy(ws_hbm.at[pl.ds(row0, _TSUB)], w0s_v)
        pltpu.sync_copy(ws_hbm.at[pl.ds(N_TOK + row0, _TSUB)], w1s_v)
        pltpu.async_copy(y_hbm.at[i0_v], a_v, sem).wait()
        pltpu.async_copy(y_hbm.at[i1_v], b_v, sem).wait()

        def row(r, _):
            s0 = w0s_v[r, :]
            s1 = w1s_v[r, :]
            for jj in range(D // 16):
                o_v[r, pl.ds(jj * 16, 16)] = (
                    s0 * a_v[r, pl.ds(jj * 16, 16)]
                    + s1 * b_v[r, pl.ds(jj * 16, 16)])
            return 0

        lax.fori_loop(0, _TSUB, row, 0)
        pltpu.sync_copy(o_v, out_hbm.at[pl.ds(row0, _TSUB)])


def _sc_combine(y, d_mat, w_splat):
    mesh = plsc.VectorSubcoreMesh(core_axis_name="c", subcore_axis_name="s", num_cores=_NC, num_subcores=_NS)
    f = pl.kernel(
        _sc_combine_body,
        out_type=jax.ShapeDtypeStruct((N_TOK, D), jnp.float32),
        mesh=mesh,
        scratch_types=[
            pltpu.VMEM((_TSUB,), jnp.int32),
            pltpu.VMEM((_TSUB,), jnp.int32),
            pltpu.VMEM((_TSUB, 16), jnp.float32),
            pltpu.VMEM((_TSUB, 16), jnp.float32),
            pltpu.VMEM((_TSUB, D), jnp.float32),
            pltpu.VMEM((_TSUB, D), jnp.float32),
            pltpu.VMEM((_TSUB, D), jnp.float32),
            pltpu.SemaphoreType.DMA,
        ],
    )
    return f(y, d_mat, w_splat)


def kernel(hidden_states, gate_weight, gate_up_proj, down_proj, similarity_matrix):
    B, S, Dm = hidden_states.shape
    x = hidden_states.reshape(-1, Dm)
    d, w_splat, bmap, bexp = _routing(x, gate_weight, similarity_matrix)
    d_mat = d.reshape(N_CHUNK, CHUNK)
    xs = _sc_scatter(x, d_mat)
    y = _ragged_gemm(bmap, bexp, xs, gate_up_proj.astype(jnp.bfloat16),
                     down_proj.astype(jnp.bfloat16))
    out = _sc_combine(y, d_mat, w_splat)
    return out.reshape(B, S, Dm)


# speedup vs baseline: 1.0539x; 1.0539x over previous
import functools

import jax
import jax.numpy as jnp
from jax import lax
from jax.experimental import pallas as pl
from jax.experimental.pallas import tpu as pltpu
from jax.experimental.pallas import tpu_sc as plsc

N_EXP = 8
D = 1024
DFF = 512
N_TOK = 2048
N_PAIR = 2 * N_TOK
BT = 256
G = N_PAIR // BT + N_EXP
P_ROWS = G * BT
CHUNK = 64
N_CHUNK = N_PAIR // CHUNK
NEG = -3.0e38


def _argmax_lanes(v, iota_row):
    m = jnp.max(v, axis=-1, keepdims=True)
    return jnp.min(jnp.where(v == m, iota_row, N_EXP), axis=-1, keepdims=True), m


def _routing_body(x_ref, gw_ref, sim_ref, tri_ref, d_ref, w_ref, bmap_ref,
                  bexp_ref):
    x = x_ref[...]
    gw = gw_ref[...]
    logits = lax.dot_general(x, gw, (((1,), (1,)), ((), ())),
                             preferred_element_type=jnp.float32)
    m = jnp.max(logits, axis=-1, keepdims=True)
    e = jnp.exp(logits - m)
    probs = e / jnp.sum(e, axis=-1, keepdims=True)

    iota_row = lax.broadcasted_iota(jnp.int32, (N_TOK, N_EXP), 1)
    i1, v1 = _argmax_lanes(probs, iota_row)
    oh1 = (iota_row == i1)
    probs2 = jnp.where(oh1, NEG, probs)
    i2, v2 = _argmax_lanes(probs2, iota_row)
    oh2 = (iota_row == i2)
    denom = jnp.maximum(v1 + v2, 1e-12)
    w1 = v1 / denom
    w2 = v2 / denom

    mask_col = jnp.max(oh1.astype(jnp.float32), axis=0, keepdims=True)
    sim = sim_ref[...]
    iota_r8 = lax.broadcasted_iota(jnp.int32, (N_EXP, N_EXP), 1)
    iota_c8 = lax.broadcasted_iota(jnp.int32, (N_EXP, N_EXP), 0)
    eye = (iota_r8 == iota_c8)
    sim_masked = jnp.where(mask_col > 0.5, sim, NEG)
    best_sim = jnp.max(sim_masked, axis=-1, keepdims=True)
    best_j = jnp.min(jnp.where(sim_masked == best_sim, iota_r8, N_EXP),
                     axis=-1, keepdims=True)
    mask_row = jnp.max(jnp.where(eye, jnp.broadcast_to(mask_col, (N_EXP, N_EXP)),
                                 0.0), axis=-1, keepdims=True)
    reroute = (mask_row < 0.5) & (best_sim >= 0.5)
    ident = lax.broadcasted_iota(jnp.int32, (N_EXP, 1), 0)
    emap = jnp.where(reroute, best_j, ident)
    perm = (emap == iota_r8).astype(jnp.float32)

    pre = w1 * oh1.astype(jnp.float32) + w2 * oh2.astype(jnp.float32)
    rw = lax.dot_general(pre, perm, (((1,), (0,)), ((), ())),
                         preferred_element_type=jnp.float32)

    f1, u1 = _argmax_lanes(rw, iota_row)
    ohf1 = (iota_row == f1)
    rwm = jnp.where(ohf1, -1.0, rw)
    f2, u2 = _argmax_lanes(rwm, iota_row)
    ohf2 = (iota_row == f2)

    w_ref[:N_TOK, :] = jnp.broadcast_to(u1, (N_TOK, 16))
    w_ref[N_TOK:, :] = jnp.broadcast_to(u2, (N_TOK, 16))

    TB = 1024
    n_blk = N_PAIR // TB
    tri = tri_ref[...]
    o_blks, c_blks, carries = [], [], []
    car = jnp.zeros((1, N_EXP), jnp.float32)
    for b in range(n_blk):
        if b < n_blk // 2:
            o_blk = ohf1[b * TB:(b + 1) * TB, :].astype(jnp.float32)
        else:
            o_blk = ohf2[(b - n_blk // 2) * TB:(b - n_blk // 2 + 1) * TB,
                         :].astype(jnp.float32)
        c = lax.dot_general(tri, o_blk, (((1,), (0,)), ((), ())),
                            preferred_element_type=jnp.float32)
        o_blks.append(o_blk)
        c_blks.append(c)
        carries.append(car)
        car = car + c[TB - 1:TB, :]

    counts = car
    nb = jnp.floor((counts + float(BT - 1)) * (1.0 / BT))
    iota_u8 = lax.broadcasted_iota(jnp.int32, (N_EXP, N_EXP), 0)
    u8 = (iota_u8 < iota_r8).astype(jnp.float32)
    excl = lax.dot_general(nb, u8, (((1,), (0,)), ((), ())),
                           preferred_element_type=jnp.float32)
    pad_base = excl * float(BT)
    total = jnp.sum(nb)
    cumnext = excl + nb

    for b in range(n_blk):
        inc = c_blks[b] + carries[b]
        rank = jnp.sum(inc * o_blks[b], axis=-1, keepdims=True) - 1.0
        pb = jnp.sum(pad_base * o_blks[b], axis=-1, keepdims=True)
        d_ref[b * TB:(b + 1) * TB, :] = (pb + rank).astype(jnp.int32)

    gcol = lax.broadcasted_iota(jnp.int32, (G, 1), 0).astype(jnp.float32)
    bmapf = jnp.minimum(gcol, total - 1.0)
    bexp = jnp.sum((jnp.broadcast_to(cumnext, (G, N_EXP)) <= bmapf)
                   .astype(jnp.int32), axis=-1, keepdims=True)
    bmap_ref[...] = bmapf.astype(jnp.int32)
    bexp_ref[...] = bexp


def _routing(x, gate_weight, sim, tri):
    return pl.pallas_call(
        _routing_body,
        out_shape=(
            jax.ShapeDtypeStruct((N_PAIR, 1), jnp.int32),
            jax.ShapeDtypeStruct((N_PAIR, 16), jnp.float32),
            jax.ShapeDtypeStruct((G, 1), jnp.int32),
            jax.ShapeDtypeStruct((G, 1), jnp.int32),
        ),
        in_specs=[
            pl.BlockSpec((N_TOK, D), lambda: (0, 0)),
            pl.BlockSpec((N_EXP, D), lambda: (0, 0)),
            pl.BlockSpec((N_EXP, N_EXP), lambda: (0, 0)),
            pl.BlockSpec((1024, 1024), lambda: (0, 0)),
        ],
        out_specs=(
            pl.BlockSpec((N_PAIR, 1), lambda: (0, 0)),
            pl.BlockSpec((N_PAIR, 16), lambda: (0, 0)),
            pl.BlockSpec((G, 1), lambda: (0, 0)),
            pl.BlockSpec((G, 1), lambda: (0, 0)),
        ),
    )(x, gate_weight, sim, tri)


_NC = 2
_NS = 16
_NW = _NC * _NS


def _sc_scatter_body(x_hbm, dmat_hbm, xs_hbm, idx_v, rows_v, sem):
    wid = lax.axis_index("s") * _NC + lax.axis_index("c")
    for half in range(2):
        j = wid + half * _NW
        t0 = wid * CHUNK
        pltpu.sync_copy(dmat_hbm.at[j], idx_v)
        pltpu.sync_copy(x_hbm.at[pl.ds(t0, CHUNK)], rows_v)
        pltpu.async_copy(rows_v, xs_hbm.at[idx_v], sem).wait()


def _sc_scatter(x, d_mat):
    mesh = plsc.VectorSubcoreMesh(core_axis_name="c", subcore_axis_name="s", num_cores=_NC, num_subcores=_NS)
    f = pl.kernel(
        _sc_scatter_body,
        out_type=jax.ShapeDtypeStruct((P_ROWS, D), jnp.float32),
        mesh=mesh,
        scratch_types=[
            pltpu.VMEM((CHUNK,), jnp.int32),
            pltpu.VMEM((CHUNK, D), jnp.float32),
            pltpu.SemaphoreType.DMA,
        ],
    )
    return f(x, d_mat)


def _gemm_body(bmap_ref, bexp_ref, xs_ref, gup_ref, down_ref, y_ref):
    g = pl.program_id(0)

    @pl.when(bmap_ref[g, 0] == g)
    def _():
        xs = xs_ref[...]
        gu = lax.dot_general(xs, gup_ref[0], (((1,), (1,)), ((), ())),
                             preferred_element_type=jnp.float32)
        gate = gu[:, :DFF]
        up = gu[:, DFF:]
        h = gate * jax.nn.sigmoid(gate) * up
        y_ref[...] = lax.dot_general(h, down_ref[0], (((1,), (1,)), ((), ())),
                                     preferred_element_type=jnp.float32)


def _ragged_gemm(bmap, bexp, xs, gate_up_proj, down_proj):
    grid_spec = pltpu.PrefetchScalarGridSpec(
        num_scalar_prefetch=2,
        grid=(G,),
        in_specs=[
            pl.BlockSpec((BT, D), lambda g, bm, be: (bm[g, 0], 0)),
            pl.BlockSpec((1, 2 * DFF, D), lambda g, bm, be: (be[g, 0], 0, 0)),
            pl.BlockSpec((1, D, DFF), lambda g, bm, be: (be[g, 0], 0, 0)),
        ],
        out_specs=pl.BlockSpec((BT, D), lambda g, bm, be: (bm[g, 0], 0)),
    )
    return pl.pallas_call(
        _gemm_body,
        grid_spec=grid_spec,
        out_shape=jax.ShapeDtypeStruct((P_ROWS, D), jnp.float32),
    )(bmap, bexp, xs, gate_up_proj, down_proj)


_TSUB = 16


def _sc_combine_body(y_hbm, dmat_hbm, ws_hbm, out_hbm,
                     i0_v, i1_v, w0s_v, w1s_v, a_v, b_v, o_v, sem):
    wid = lax.axis_index("s") * _NC + lax.axis_index("c")
    for s in range(CHUNK // _TSUB):
        col = s * _TSUB
        row0 = wid * CHUNK + col
        pltpu.sync_copy(dmat_hbm.at[wid, pl.ds(col, _TSUB)], i0_v)
        pltpu.sync_copy(dmat_hbm.at[wid + _NW, pl.ds(col, _TSUB)], i1_v)
        pltpu.sync_copy(ws_hbm.at[pl.ds(row0, _TSUB)], w0s_v)
        pltpu.sync_copy(ws_hbm.at[pl.ds(N_TOK + row0, _TSUB)], w1s_v)
        pltpu.async_copy(y_hbm.at[i0_v], a_v, sem).wait()
        pltpu.async_copy(y_hbm.at[i1_v], b_v, sem).wait()

        def row(r, _):
            s0 = w0s_v[r, :]
            s1 = w1s_v[r, :]
            for jj in range(D // 16):
                o_v[r, pl.ds(jj * 16, 16)] = (
                    s0 * a_v[r, pl.ds(jj * 16, 16)]
                    + s1 * b_v[r, pl.ds(jj * 16, 16)])
            return 0

        lax.fori_loop(0, _TSUB, row, 0)
        pltpu.sync_copy(o_v, out_hbm.at[pl.ds(row0, _TSUB)])


def _sc_combine(y, d_mat, w_splat):
    mesh = plsc.VectorSubcoreMesh(core_axis_name="c", subcore_axis_name="s", num_cores=_NC, num_subcores=_NS)
    f = pl.kernel(
        _sc_combine_body,
        out_type=jax.ShapeDtypeStruct((N_TOK, D), jnp.float32),
        mesh=mesh,
        scratch_types=[
            pltpu.VMEM((_TSUB,), jnp.int32),
            pltpu.VMEM((_TSUB,), jnp.int32),
            pltpu.VMEM((_TSUB, 16), jnp.float32),
            pltpu.VMEM((_TSUB, 16), jnp.float32),
            pltpu.VMEM((_TSUB, D), jnp.float32),
            pltpu.VMEM((_TSUB, D), jnp.float32),
            pltpu.VMEM((_TSUB, D), jnp.float32),
            pltpu.SemaphoreType.DMA,
        ],
    )
    return f(y, d_mat, w_splat)


def kernel(hidden_states, gate_weight, gate_up_proj, down_proj, similarity_matrix):
    B, S, Dm = hidden_states.shape
    x = hidden_states.reshape(-1, Dm)
    tri = jnp.tril(jnp.ones((1024, 1024), jnp.float32))
    d, w_splat, bmap, bexp = _routing(x, gate_weight, similarity_matrix, tri)
    d_mat = d.reshape(N_CHUNK, CHUNK)
    xs = _sc_scatter(x, d_mat)
    y = _ragged_gemm(bmap, bexp, xs, gate_up_proj, down_proj)
    out = _sc_combine(y, d_mat, w_splat)
    return out.reshape(B, S, Dm)


# speedup vs baseline: 1.0968x; 1.0407x over previous
import functools

import jax
import jax.numpy as jnp
from jax import lax
from jax.experimental import pallas as pl
from jax.experimental.pallas import tpu as pltpu
from jax.experimental.pallas import tpu_sc as plsc

N_EXP = 8
D = 1024
DFF = 512
N_TOK = 2048
N_PAIR = 2 * N_TOK
BT = 256
G = N_PAIR // BT + N_EXP
P_ROWS = G * BT
CHUNK = 64
N_CHUNK = N_PAIR // CHUNK
NEG = -3.0e38


def _argmax_lanes(v, iota_row):
    m = jnp.max(v, axis=-1, keepdims=True)
    return jnp.min(jnp.where(v == m, iota_row, N_EXP), axis=-1, keepdims=True), m


def _routing_body(x_ref, gw_ref, sim_ref, tri_ref, d_ref, w_ref, bmap_ref,
                  bexp_ref):
    x = x_ref[...]
    gw = gw_ref[...]
    logits = lax.dot_general(x, gw, (((1,), (1,)), ((), ())),
                             preferred_element_type=jnp.float32)
    m = jnp.max(logits, axis=-1, keepdims=True)
    e = jnp.exp(logits - m)
    probs = e / jnp.sum(e, axis=-1, keepdims=True)

    iota_row = lax.broadcasted_iota(jnp.int32, (N_TOK, N_EXP), 1)
    i1, v1 = _argmax_lanes(probs, iota_row)
    oh1 = (iota_row == i1)
    probs2 = jnp.where(oh1, NEG, probs)
    i2, v2 = _argmax_lanes(probs2, iota_row)
    oh2 = (iota_row == i2)
    denom = jnp.maximum(v1 + v2, 1e-12)
    w1 = v1 / denom
    w2 = v2 / denom

    mask_col = jnp.max(oh1.astype(jnp.float32), axis=0, keepdims=True)
    sim = sim_ref[...]
    iota_r8 = lax.broadcasted_iota(jnp.int32, (N_EXP, N_EXP), 1)
    iota_c8 = lax.broadcasted_iota(jnp.int32, (N_EXP, N_EXP), 0)
    eye = (iota_r8 == iota_c8)
    sim_masked = jnp.where(mask_col > 0.5, sim, NEG)
    best_sim = jnp.max(sim_masked, axis=-1, keepdims=True)
    best_j = jnp.min(jnp.where(sim_masked == best_sim, iota_r8, N_EXP),
                     axis=-1, keepdims=True)
    mask_row = jnp.max(jnp.where(eye, jnp.broadcast_to(mask_col, (N_EXP, N_EXP)),
                                 0.0), axis=-1, keepdims=True)
    reroute = (mask_row < 0.5) & (best_sim >= 0.5)
    ident = lax.broadcasted_iota(jnp.int32, (N_EXP, 1), 0)
    emap = jnp.where(reroute, best_j, ident)
    perm = (emap == iota_r8).astype(jnp.float32)

    pre = w1 * oh1.astype(jnp.float32) + w2 * oh2.astype(jnp.float32)
    rw = lax.dot_general(pre, perm, (((1,), (0,)), ((), ())),
                         preferred_element_type=jnp.float32)

    f1, u1 = _argmax_lanes(rw, iota_row)
    ohf1 = (iota_row == f1)
    rwm = jnp.where(ohf1, -1.0, rw)
    f2, u2 = _argmax_lanes(rwm, iota_row)
    ohf2 = (iota_row == f2)

    w_ref[:N_TOK, :] = jnp.broadcast_to(u1, (N_TOK, 16))
    w_ref[N_TOK:, :] = jnp.broadcast_to(u2, (N_TOK, 16))

    TB = 256
    n_blk = N_PAIR // TB
    tri = tri_ref[...]
    o_blks, c_blks, carries = [], [], []
    car = jnp.zeros((1, N_EXP), jnp.float32)
    for b in range(n_blk):
        if b < n_blk // 2:
            o_blk = ohf1[b * TB:(b + 1) * TB, :].astype(jnp.float32)
        else:
            o_blk = ohf2[(b - n_blk // 2) * TB:(b - n_blk // 2 + 1) * TB,
                         :].astype(jnp.float32)
        c = lax.dot_general(tri, o_blk, (((1,), (0,)), ((), ())),
                            preferred_element_type=jnp.float32)
        o_blks.append(o_blk)
        c_blks.append(c)
        carries.append(car)
        car = car + c[TB - 1:TB, :]

    counts = car
    nb = jnp.floor((counts + float(BT - 1)) * (1.0 / BT))
    iota_u8 = lax.broadcasted_iota(jnp.int32, (N_EXP, N_EXP), 0)
    u8 = (iota_u8 < iota_r8).astype(jnp.float32)
    excl = lax.dot_general(nb, u8, (((1,), (0,)), ((), ())),
                           preferred_element_type=jnp.float32)
    pad_base = excl * float(BT)
    total = jnp.sum(nb)
    cumnext = excl + nb

    for b in range(n_blk):
        inc = c_blks[b] + carries[b]
        rank = jnp.sum(inc * o_blks[b], axis=-1, keepdims=True) - 1.0
        pb = jnp.sum(pad_base * o_blks[b], axis=-1, keepdims=True)
        d_ref[b * TB:(b + 1) * TB, :] = (pb + rank).astype(jnp.int32)

    gcol = lax.broadcasted_iota(jnp.int32, (G, 1), 0).astype(jnp.float32)
    bmapf = jnp.minimum(gcol, total - 1.0)
    bexp = jnp.sum((jnp.broadcast_to(cumnext, (G, N_EXP)) <= bmapf)
                   .astype(jnp.int32), axis=-1, keepdims=True)
    bmap_ref[...] = bmapf.astype(jnp.int32)
    bexp_ref[...] = bexp


def _routing(x, gate_weight, sim, tri):
    return pl.pallas_call(
        _routing_body,
        out_shape=(
            jax.ShapeDtypeStruct((N_PAIR, 1), jnp.int32),
            jax.ShapeDtypeStruct((N_PAIR, 16), jnp.float32),
            jax.ShapeDtypeStruct((G, 1), jnp.int32),
            jax.ShapeDtypeStruct((G, 1), jnp.int32),
        ),
        in_specs=[
            pl.BlockSpec((N_TOK, D), lambda: (0, 0)),
            pl.BlockSpec((N_EXP, D), lambda: (0, 0)),
            pl.BlockSpec((N_EXP, N_EXP), lambda: (0, 0)),
            pl.BlockSpec((256, 256), lambda: (0, 0)),
        ],
        out_specs=(
            pl.BlockSpec((N_PAIR, 1), lambda: (0, 0)),
            pl.BlockSpec((N_PAIR, 16), lambda: (0, 0)),
            pl.BlockSpec((G, 1), lambda: (0, 0)),
            pl.BlockSpec((G, 1), lambda: (0, 0)),
        ),
    )(x, gate_weight, sim, tri)


_NC = 2
_NS = 16
_NW = _NC * _NS


def _sc_scatter_body(x_hbm, dmat_hbm, xs_hbm, idx_v, rows_v, sem):
    wid = lax.axis_index("s") * _NC + lax.axis_index("c")
    for half in range(2):
        j = wid + half * _NW
        t0 = wid * CHUNK
        pltpu.sync_copy(dmat_hbm.at[j], idx_v)
        pltpu.sync_copy(x_hbm.at[pl.ds(t0, CHUNK)], rows_v)
        pltpu.async_copy(rows_v, xs_hbm.at[idx_v], sem).wait()


def _sc_scatter(x, d_mat):
    mesh = plsc.VectorSubcoreMesh(core_axis_name="c", subcore_axis_name="s", num_cores=_NC, num_subcores=_NS)
    f = pl.kernel(
        _sc_scatter_body,
        out_type=jax.ShapeDtypeStruct((P_ROWS, D), jnp.float32),
        mesh=mesh,
        scratch_types=[
            pltpu.VMEM((CHUNK,), jnp.int32),
            pltpu.VMEM((CHUNK, D), jnp.float32),
            pltpu.SemaphoreType.DMA,
        ],
    )
    return f(x, d_mat)


def _gemm_body(bmap_ref, bexp_ref, xs_ref, gup_ref, down_ref, y_ref):
    g = pl.program_id(0)

    @pl.when(bmap_ref[g, 0] == g)
    def _():
        xs = xs_ref[...]
        gu = lax.dot_general(xs, gup_ref[0], (((1,), (1,)), ((), ())),
                             preferred_element_type=jnp.float32)
        gate = gu[:, :DFF]
        up = gu[:, DFF:]
        h = gate * jax.nn.sigmoid(gate) * up
        y_ref[...] = lax.dot_general(h, down_ref[0], (((1,), (1,)), ((), ())),
                                     preferred_element_type=jnp.float32)


def _ragged_gemm(bmap, bexp, xs, gate_up_proj, down_proj):
    grid_spec = pltpu.PrefetchScalarGridSpec(
        num_scalar_prefetch=2,
        grid=(G,),
        in_specs=[
            pl.BlockSpec((BT, D), lambda g, bm, be: (bm[g, 0], 0)),
            pl.BlockSpec((1, 2 * DFF, D), lambda g, bm, be: (be[g, 0], 0, 0)),
            pl.BlockSpec((1, D, DFF), lambda g, bm, be: (be[g, 0], 0, 0)),
        ],
        out_specs=pl.BlockSpec((BT, D), lambda g, bm, be: (bm[g, 0], 0)),
    )
    return pl.pallas_call(
        _gemm_body,
        grid_spec=grid_spec,
        out_shape=jax.ShapeDtypeStruct((P_ROWS, D), jnp.float32),
    )(bmap, bexp, xs, gate_up_proj, down_proj)


_TSUB = 16


def _sc_combine_body(y_hbm, dmat_hbm, ws_hbm, out_hbm,
                     i0_v, i1_v, w0s_v, w1s_v, a_v, b_v, o_v, sem):
    wid = lax.axis_index("s") * _NC + lax.axis_index("c")
    for s in range(CHUNK // _TSUB):
        col = s * _TSUB
        row0 = wid * CHUNK + col
        pltpu.sync_copy(dmat_hbm.at[wid, pl.ds(col, _TSUB)], i0_v)
        pltpu.sync_copy(dmat_hbm.at[wid + _NW, pl.ds(col, _TSUB)], i1_v)
        pltpu.sync_copy(ws_hbm.at[pl.ds(row0, _TSUB)], w0s_v)
        pltpu.sync_copy(ws_hbm.at[pl.ds(N_TOK + row0, _TSUB)], w1s_v)
        pltpu.async_copy(y_hbm.at[i0_v], a_v, sem).wait()
        pltpu.async_copy(y_hbm.at[i1_v], b_v, sem).wait()

        def row(r, _):
            s0 = w0s_v[r, :]
            s1 = w1s_v[r, :]
            for jj in range(D // 16):
                o_v[r, pl.ds(jj * 16, 16)] = (
                    s0 * a_v[r, pl.ds(jj * 16, 16)]
                    + s1 * b_v[r, pl.ds(jj * 16, 16)])
            return 0

        lax.fori_loop(0, _TSUB, row, 0)
        pltpu.sync_copy(o_v, out_hbm.at[pl.ds(row0, _TSUB)])


def _sc_combine(y, d_mat, w_splat):
    mesh = plsc.VectorSubcoreMesh(core_axis_name="c", subcore_axis_name="s", num_cores=_NC, num_subcores=_NS)
    f = pl.kernel(
        _sc_combine_body,
        out_type=jax.ShapeDtypeStruct((N_TOK, D), jnp.float32),
        mesh=mesh,
        scratch_types=[
            pltpu.VMEM((_TSUB,), jnp.int32),
            pltpu.VMEM((_TSUB,), jnp.int32),
            pltpu.VMEM((_TSUB, 16), jnp.float32),
            pltpu.VMEM((_TSUB, 16), jnp.float32),
            pltpu.VMEM((_TSUB, D), jnp.float32),
            pltpu.VMEM((_TSUB, D), jnp.float32),
            pltpu.VMEM((_TSUB, D), jnp.float32),
            pltpu.SemaphoreType.DMA,
        ],
    )
    return f(y, d_mat, w_splat)


def kernel(hidden_states, gate_weight, gate_up_proj, down_proj, similarity_matrix):
    B, S, Dm = hidden_states.shape
    x = hidden_states.reshape(-1, Dm)
    tri = jnp.tril(jnp.ones((256, 256), jnp.float32))
    d, w_splat, bmap, bexp = _routing(x, gate_weight, similarity_matrix, tri)
    d_mat = d.reshape(N_CHUNK, CHUNK)
    xs = _sc_scatter(x, d_mat)
    y = _ragged_gemm(bmap, bexp, xs, gate_up_proj, down_proj)
    out = _sc_combine(y, d_mat, w_splat)
    return out.reshape(B, S, Dm)


# speedup vs baseline: 1.2268x; 1.1185x over previous
import functools

import jax
import jax.numpy as jnp
from jax import lax
from jax.experimental import pallas as pl
from jax.experimental.pallas import tpu as pltpu
from jax.experimental.pallas import tpu_sc as plsc

N_EXP = 8
D = 1024
DFF = 512
N_TOK = 2048
N_PAIR = 2 * N_TOK
BT = 256
G = N_PAIR // BT + N_EXP
P_ROWS = G * BT
CHUNK = 64
N_CHUNK = N_PAIR // CHUNK
NEG = -3.0e38


def _argmax_lanes(v, iota_row):
    m = jnp.max(v, axis=-1, keepdims=True)
    return jnp.min(jnp.where(v == m, iota_row, N_EXP), axis=-1, keepdims=True), m


def _routing_body(x_ref, gw_ref, sim_ref, tri_ref, d_ref, w_ref, bmap_ref,
                  bexp_ref):
    x = x_ref[...]
    gw = gw_ref[...]
    logits = lax.dot_general(x, gw, (((1,), (1,)), ((), ())),
                             preferred_element_type=jnp.float32)
    m = jnp.max(logits, axis=-1, keepdims=True)
    e = jnp.exp(logits - m)
    probs = e / jnp.sum(e, axis=-1, keepdims=True)

    iota_row = lax.broadcasted_iota(jnp.int32, (N_TOK, N_EXP), 1)
    i1, v1 = _argmax_lanes(probs, iota_row)
    oh1 = (iota_row == i1)
    probs2 = jnp.where(oh1, NEG, probs)
    i2, v2 = _argmax_lanes(probs2, iota_row)
    oh2 = (iota_row == i2)
    denom = jnp.maximum(v1 + v2, 1e-12)
    w1 = v1 / denom
    w2 = v2 / denom

    mask_col = jnp.max(oh1.astype(jnp.float32), axis=0, keepdims=True)
    sim = sim_ref[...]
    iota_r8 = lax.broadcasted_iota(jnp.int32, (N_EXP, N_EXP), 1)
    iota_c8 = lax.broadcasted_iota(jnp.int32, (N_EXP, N_EXP), 0)
    eye = (iota_r8 == iota_c8)
    sim_masked = jnp.where(mask_col > 0.5, sim, NEG)
    best_sim = jnp.max(sim_masked, axis=-1, keepdims=True)
    best_j = jnp.min(jnp.where(sim_masked == best_sim, iota_r8, N_EXP),
                     axis=-1, keepdims=True)
    mask_row = jnp.max(jnp.where(eye, jnp.broadcast_to(mask_col, (N_EXP, N_EXP)),
                                 0.0), axis=-1, keepdims=True)
    reroute = (mask_row < 0.5) & (best_sim >= 0.5)
    ident = lax.broadcasted_iota(jnp.int32, (N_EXP, 1), 0)
    emap = jnp.where(reroute, best_j, ident)
    perm = (emap == iota_r8).astype(jnp.float32)

    pre = w1 * oh1.astype(jnp.float32) + w2 * oh2.astype(jnp.float32)
    rw = lax.dot_general(pre, perm, (((1,), (0,)), ((), ())),
                         preferred_element_type=jnp.float32)

    f1, u1 = _argmax_lanes(rw, iota_row)
    ohf1 = (iota_row == f1)
    rwm = jnp.where(ohf1, -1.0, rw)
    f2, u2 = _argmax_lanes(rwm, iota_row)
    ohf2 = (iota_row == f2)

    w_ref[:N_TOK, :] = jnp.broadcast_to(u1, (N_TOK, 16))
    w_ref[N_TOK:, :] = jnp.broadcast_to(u2, (N_TOK, 16))

    TB = 256
    n_blk = N_PAIR // TB
    tri = tri_ref[...]
    o_blks, c_blks, carries = [], [], []
    car = jnp.zeros((1, N_EXP), jnp.float32)
    for b in range(n_blk):
        if b < n_blk // 2:
            o_blk = ohf1[b * TB:(b + 1) * TB, :].astype(jnp.float32)
        else:
            o_blk = ohf2[(b - n_blk // 2) * TB:(b - n_blk // 2 + 1) * TB,
                         :].astype(jnp.float32)
        c = lax.dot_general(tri, o_blk, (((1,), (0,)), ((), ())),
                            preferred_element_type=jnp.float32)
        o_blks.append(o_blk)
        c_blks.append(c)
        carries.append(car)
        car = car + c[TB - 1:TB, :]

    counts = car
    nb = jnp.floor((counts + float(BT - 1)) * (1.0 / BT))
    iota_u8 = lax.broadcasted_iota(jnp.int32, (N_EXP, N_EXP), 0)
    u8 = (iota_u8 < iota_r8).astype(jnp.float32)
    excl = lax.dot_general(nb, u8, (((1,), (0,)), ((), ())),
                           preferred_element_type=jnp.float32)
    pad_base = excl * float(BT)
    total = jnp.sum(nb)
    cumnext = excl + nb

    for b in range(n_blk):
        inc = c_blks[b] + carries[b]
        rank = jnp.sum(inc * o_blks[b], axis=-1, keepdims=True) - 1.0
        pb = jnp.sum(pad_base * o_blks[b], axis=-1, keepdims=True)
        d_ref[b * TB:(b + 1) * TB, :] = (pb + rank).astype(jnp.int32)

    gcol = lax.broadcasted_iota(jnp.int32, (G, 1), 0).astype(jnp.float32)
    bmapf = jnp.minimum(gcol, total - 1.0)
    bexp = jnp.sum((jnp.broadcast_to(cumnext, (G, N_EXP)) <= bmapf)
                   .astype(jnp.int32), axis=-1, keepdims=True)
    bmap_ref[...] = bmapf.astype(jnp.int32)
    bexp_ref[...] = bexp


def _routing(x, gate_weight, sim, tri):
    return pl.pallas_call(
        _routing_body,
        out_shape=(
            jax.ShapeDtypeStruct((N_PAIR, 1), jnp.int32),
            jax.ShapeDtypeStruct((N_PAIR, 16), jnp.float32),
            jax.ShapeDtypeStruct((G, 1), jnp.int32),
            jax.ShapeDtypeStruct((G, 1), jnp.int32),
        ),
        in_specs=[
            pl.BlockSpec((N_TOK, D), lambda: (0, 0)),
            pl.BlockSpec((N_EXP, D), lambda: (0, 0)),
            pl.BlockSpec((N_EXP, N_EXP), lambda: (0, 0)),
            pl.BlockSpec((256, 256), lambda: (0, 0)),
        ],
        out_specs=(
            pl.BlockSpec((N_PAIR, 1), lambda: (0, 0)),
            pl.BlockSpec((N_PAIR, 16), lambda: (0, 0)),
            pl.BlockSpec((G, 1), lambda: (0, 0)),
            pl.BlockSpec((G, 1), lambda: (0, 0)),
        ),
    )(x, gate_weight, sim, tri)


_NC = 2
_NS = 16
_NW = _NC * _NS


def _sc_scatter_body(x_hbm, dmat_hbm, xs_hbm, idx0_v, idx1_v, rows0_v,
                     rsem, ssem):
    wid = lax.axis_index("s") * _NC + lax.axis_index("c")
    t0 = wid * CHUNK
    pltpu.sync_copy(dmat_hbm.at[wid], idx0_v)
    pltpu.sync_copy(dmat_hbm.at[wid + _NW], idx1_v)
    rows_cp = pltpu.async_copy(x_hbm.at[pl.ds(t0, CHUNK)], rows0_v, rsem)
    rows_cp.wait()
    sc0 = pltpu.async_copy(rows0_v, xs_hbm.at[idx0_v], ssem)
    sc1 = pltpu.async_copy(rows0_v, xs_hbm.at[idx1_v], ssem)
    sc0.wait()
    sc1.wait()


def _sc_scatter(x, d_mat):
    mesh = plsc.VectorSubcoreMesh(core_axis_name="c", subcore_axis_name="s", num_cores=_NC, num_subcores=_NS)
    f = pl.kernel(
        _sc_scatter_body,
        out_type=jax.ShapeDtypeStruct((P_ROWS, D), jnp.float32),
        mesh=mesh,
        scratch_types=[
            pltpu.VMEM((CHUNK,), jnp.int32),
            pltpu.VMEM((CHUNK,), jnp.int32),
            pltpu.VMEM((CHUNK, D), jnp.float32),
            pltpu.SemaphoreType.DMA,
            pltpu.SemaphoreType.DMA,
        ],
    )
    return f(x, d_mat)


def _gemm_body(bmap_ref, bexp_ref, xs_ref, gup_ref, down_ref, y_ref):
    g = pl.program_id(0)

    @pl.when(bmap_ref[g, 0] == g)
    def _():
        xs = xs_ref[...]
        gu = lax.dot_general(xs, gup_ref[0], (((1,), (1,)), ((), ())),
                             preferred_element_type=jnp.float32)
        gate = gu[:, :DFF]
        up = gu[:, DFF:]
        h = gate * jax.nn.sigmoid(gate) * up
        y_ref[...] = lax.dot_general(h, down_ref[0], (((1,), (1,)), ((), ())),
                                     preferred_element_type=jnp.float32)


def _ragged_gemm(bmap, bexp, xs, gate_up_proj, down_proj):
    grid_spec = pltpu.PrefetchScalarGridSpec(
        num_scalar_prefetch=2,
        grid=(G,),
        in_specs=[
            pl.BlockSpec((BT, D), lambda g, bm, be: (bm[g, 0], 0)),
            pl.BlockSpec((1, 2 * DFF, D), lambda g, bm, be: (be[g, 0], 0, 0)),
            pl.BlockSpec((1, D, DFF), lambda g, bm, be: (be[g, 0], 0, 0)),
        ],
        out_specs=pl.BlockSpec((BT, D), lambda g, bm, be: (bm[g, 0], 0)),
    )
    return pl.pallas_call(
        _gemm_body,
        grid_spec=grid_spec,
        out_shape=jax.ShapeDtypeStruct((P_ROWS, D), jnp.float32),
    )(bmap, bexp, xs, gate_up_proj, down_proj)


_TSUB = 16


_N_SUB = CHUNK // _TSUB


def _sc_combine_body(y_hbm, dmat_hbm, ws_hbm, out_hbm,
                     i0a, i0b, i1a, i1b, w0a, w0b, w1a, w1b,
                     aa, ab, ba, bb, oa, ob,
                     gsa, gsb, osa, osb):
    wid = lax.axis_index("s") * _NC + lax.axis_index("c")
    i0 = (i0a, i0b)
    i1 = (i1a, i1b)
    w0 = (w0a, w0b)
    w1 = (w1a, w1b)
    av = (aa, ab)
    bv = (ba, bb)
    ov = (oa, ob)
    gsem = (gsa, gsb)
    osem = (osa, osb)

    def issue(s, bi):
        col = s * _TSUB
        row0 = wid * CHUNK + col
        pltpu.sync_copy(dmat_hbm.at[wid, pl.ds(col, _TSUB)], i0[bi])
        pltpu.sync_copy(dmat_hbm.at[wid + _NW, pl.ds(col, _TSUB)], i1[bi])
        pltpu.sync_copy(ws_hbm.at[pl.ds(row0, _TSUB)], w0[bi])
        pltpu.sync_copy(ws_hbm.at[pl.ds(N_TOK + row0, _TSUB)], w1[bi])
        ca = pltpu.async_copy(y_hbm.at[i0[bi]], av[bi], gsem[bi])
        cb = pltpu.async_copy(y_hbm.at[i1[bi]], bv[bi], gsem[bi])
        return ca, cb

    pend = issue(0, 0)
    out_pend = [None, None]
    for s in range(_N_SUB):
        bi = s % 2
        pend[0].wait()
        pend[1].wait()
        if s + 1 < _N_SUB:
            pend = issue(s + 1, 1 - bi)
        if out_pend[bi] is not None:
            out_pend[bi].wait()

        def row(r, _, bi=bi):
            s0 = w0[bi][r, :]
            s1 = w1[bi][r, :]
            for jj in range(D // 16):
                ov[bi][r, pl.ds(jj * 16, 16)] = (
                    s0 * av[bi][r, pl.ds(jj * 16, 16)]
                    + s1 * bv[bi][r, pl.ds(jj * 16, 16)])
            return 0

        lax.fori_loop(0, _TSUB, row, 0)
        row0 = wid * CHUNK + s * _TSUB
        out_pend[bi] = pltpu.async_copy(ov[bi], out_hbm.at[pl.ds(row0, _TSUB)],
                                        osem[bi])
    for cp in out_pend:
        if cp is not None:
            cp.wait()


def _sc_combine(y, d_mat, w_splat):
    mesh = plsc.VectorSubcoreMesh(core_axis_name="c", subcore_axis_name="s", num_cores=_NC, num_subcores=_NS)
    f = pl.kernel(
        _sc_combine_body,
        out_type=jax.ShapeDtypeStruct((N_TOK, D), jnp.float32),
        mesh=mesh,
        scratch_types=(
            [pltpu.VMEM((_TSUB,), jnp.int32)] * 4
            + [pltpu.VMEM((_TSUB, 16), jnp.float32)] * 4
            + [pltpu.VMEM((_TSUB, D), jnp.float32)] * 6
            + [pltpu.SemaphoreType.DMA] * 4
        ),
    )
    return f(y, d_mat, w_splat)


def kernel(hidden_states, gate_weight, gate_up_proj, down_proj, similarity_matrix):
    B, S, Dm = hidden_states.shape
    x = hidden_states.reshape(-1, Dm)
    tri = jnp.tril(jnp.ones((256, 256), jnp.float32))
    d, w_splat, bmap, bexp = _routing(x, gate_weight, similarity_matrix, tri)
    d_mat = d.reshape(N_CHUNK, CHUNK)
    xs = _sc_scatter(x, d_mat)
    y = _ragged_gemm(bmap, bexp, xs, gate_up_proj, down_proj)
    out = _sc_combine(y, d_mat, w_splat)
    return out.reshape(B, S, Dm)


# speedup vs baseline: 1.2351x; 1.0068x over previous
import functools

import jax
import jax.numpy as jnp
from jax import lax
from jax.experimental import pallas as pl
from jax.experimental.pallas import tpu as pltpu
from jax.experimental.pallas import tpu_sc as plsc

N_EXP = 8
D = 1024
DFF = 512
N_TOK = 2048
N_PAIR = 2 * N_TOK
BT = 256
G = N_PAIR // BT + N_EXP
P_ROWS = G * BT
CHUNK = 64
N_CHUNK = N_PAIR // CHUNK
NEG = -3.0e38


def _argmax_lanes(v, iota_row):
    m = jnp.max(v, axis=-1, keepdims=True)
    return jnp.min(jnp.where(v == m, iota_row, N_EXP), axis=-1, keepdims=True), m


def _routing_body(x_ref, gw_ref, sim_ref, tri_ref, d_ref, w_ref, bmap_ref,
                  bexp_ref):
    x = x_ref[...]
    gw = gw_ref[...]
    logits = lax.dot_general(x, gw, (((1,), (1,)), ((), ())),
                             preferred_element_type=jnp.float32)
    m = jnp.max(logits, axis=-1, keepdims=True)
    e = jnp.exp(logits - m)
    probs = e / jnp.sum(e, axis=-1, keepdims=True)

    iota_row = lax.broadcasted_iota(jnp.int32, (N_TOK, N_EXP), 1)
    i1, v1 = _argmax_lanes(probs, iota_row)
    oh1 = (iota_row == i1)
    probs2 = jnp.where(oh1, NEG, probs)
    i2, v2 = _argmax_lanes(probs2, iota_row)
    oh2 = (iota_row == i2)
    denom = jnp.maximum(v1 + v2, 1e-12)
    w1 = v1 / denom
    w2 = v2 / denom

    mask_col = jnp.max(oh1.astype(jnp.float32), axis=0, keepdims=True)
    sim = sim_ref[...]
    iota_r8 = lax.broadcasted_iota(jnp.int32, (N_EXP, N_EXP), 1)
    iota_c8 = lax.broadcasted_iota(jnp.int32, (N_EXP, N_EXP), 0)
    eye = (iota_r8 == iota_c8)
    sim_masked = jnp.where(mask_col > 0.5, sim, NEG)
    best_sim = jnp.max(sim_masked, axis=-1, keepdims=True)
    best_j = jnp.min(jnp.where(sim_masked == best_sim, iota_r8, N_EXP),
                     axis=-1, keepdims=True)
    mask_row = jnp.max(jnp.where(eye, jnp.broadcast_to(mask_col, (N_EXP, N_EXP)),
                                 0.0), axis=-1, keepdims=True)
    reroute = (mask_row < 0.5) & (best_sim >= 0.5)
    ident = lax.broadcasted_iota(jnp.int32, (N_EXP, 1), 0)
    emap = jnp.where(reroute, best_j, ident)
    perm = (emap == iota_r8).astype(jnp.float32)

    pre = w1 * oh1.astype(jnp.float32) + w2 * oh2.astype(jnp.float32)
    rw = lax.dot_general(pre, perm, (((1,), (0,)), ((), ())),
                         preferred_element_type=jnp.float32)

    f1, u1 = _argmax_lanes(rw, iota_row)
    ohf1 = (iota_row == f1)
    rwm = jnp.where(ohf1, -1.0, rw)
    f2, u2 = _argmax_lanes(rwm, iota_row)
    ohf2 = (iota_row == f2)

    w_ref[:N_TOK, :] = jnp.broadcast_to(u1, (N_TOK, 16))
    w_ref[N_TOK:, :] = jnp.broadcast_to(u2, (N_TOK, 16))

    TB = 256
    n_blk = N_PAIR // TB
    tri = tri_ref[...]
    o_blks, c_blks, carries = [], [], []
    car = jnp.zeros((1, N_EXP), jnp.float32)
    for b in range(n_blk):
        if b < n_blk // 2:
            o_blk = ohf1[b * TB:(b + 1) * TB, :].astype(jnp.float32)
        else:
            o_blk = ohf2[(b - n_blk // 2) * TB:(b - n_blk // 2 + 1) * TB,
                         :].astype(jnp.float32)
        c = lax.dot_general(tri, o_blk, (((1,), (0,)), ((), ())),
                            preferred_element_type=jnp.float32)
        o_blks.append(o_blk)
        c_blks.append(c)
        carries.append(car)
        car = car + c[TB - 1:TB, :]

    counts = car
    nb = jnp.floor((counts + float(BT - 1)) * (1.0 / BT))
    iota_u8 = lax.broadcasted_iota(jnp.int32, (N_EXP, N_EXP), 0)
    u8 = (iota_u8 < iota_r8).astype(jnp.float32)
    excl = lax.dot_general(nb, u8, (((1,), (0,)), ((), ())),
                           preferred_element_type=jnp.float32)
    pad_base = excl * float(BT)
    total = jnp.sum(nb)
    cumnext = excl + nb

    for b in range(n_blk):
        inc = c_blks[b] + carries[b]
        rank = jnp.sum(inc * o_blks[b], axis=-1, keepdims=True) - 1.0
        pb = jnp.sum(pad_base * o_blks[b], axis=-1, keepdims=True)
        d_ref[b * TB:(b + 1) * TB, :] = (pb + rank).astype(jnp.int32)

    gcol = lax.broadcasted_iota(jnp.int32, (G, 1), 0).astype(jnp.float32)
    bmapf = jnp.minimum(gcol, total - 1.0)
    bexp = jnp.sum((jnp.broadcast_to(cumnext, (G, N_EXP)) <= bmapf)
                   .astype(jnp.int32), axis=-1, keepdims=True)
    bmap_ref[...] = bmapf.astype(jnp.int32)
    bexp_ref[...] = bexp


def _routing(x, gate_weight, sim, tri):
    return pl.pallas_call(
        _routing_body,
        out_shape=(
            jax.ShapeDtypeStruct((N_PAIR, 1), jnp.int32),
            jax.ShapeDtypeStruct((N_PAIR, 16), jnp.float32),
            jax.ShapeDtypeStruct((G, 1), jnp.int32),
            jax.ShapeDtypeStruct((G, 1), jnp.int32),
        ),
        in_specs=[
            pl.BlockSpec((N_TOK, D), lambda: (0, 0)),
            pl.BlockSpec((N_EXP, D), lambda: (0, 0)),
            pl.BlockSpec((N_EXP, N_EXP), lambda: (0, 0)),
            pl.BlockSpec((256, 256), lambda: (0, 0)),
        ],
        out_specs=(
            pl.BlockSpec((N_PAIR, 1), lambda: (0, 0)),
            pl.BlockSpec((N_PAIR, 16), lambda: (0, 0)),
            pl.BlockSpec((G, 1), lambda: (0, 0)),
            pl.BlockSpec((G, 1), lambda: (0, 0)),
        ),
    )(x, gate_weight, sim, tri)


_NC = 2
_NS = 16
_NW = _NC * _NS


def _sc_scatter_body(x_hbm, dmat_hbm, xs_hbm, idx0_v, idx1_v, rows0_v,
                     rsem, ssem):
    wid = lax.axis_index("s") * _NC + lax.axis_index("c")
    t0 = wid * CHUNK
    pltpu.sync_copy(dmat_hbm.at[wid], idx0_v)
    pltpu.sync_copy(dmat_hbm.at[wid + _NW], idx1_v)
    rows_cp = pltpu.async_copy(x_hbm.at[pl.ds(t0, CHUNK)], rows0_v, rsem)
    rows_cp.wait()
    sc0 = pltpu.async_copy(rows0_v, xs_hbm.at[idx0_v], ssem)
    sc1 = pltpu.async_copy(rows0_v, xs_hbm.at[idx1_v], ssem)
    sc0.wait()
    sc1.wait()


def _sc_scatter(x, d_mat):
    mesh = plsc.VectorSubcoreMesh(core_axis_name="c", subcore_axis_name="s", num_cores=_NC, num_subcores=_NS)
    f = pl.kernel(
        _sc_scatter_body,
        out_type=jax.ShapeDtypeStruct((P_ROWS, D), jnp.float32),
        mesh=mesh,
        scratch_types=[
            pltpu.VMEM((CHUNK,), jnp.int32),
            pltpu.VMEM((CHUNK,), jnp.int32),
            pltpu.VMEM((CHUNK, D), jnp.float32),
            pltpu.SemaphoreType.DMA,
            pltpu.SemaphoreType.DMA,
        ],
    )
    return f(x, d_mat)


def _gemm_body(bmap_ref, bexp_ref, xs_ref, gup_ref, down_ref, y_ref):
    g = pl.program_id(0)

    @pl.when(bmap_ref[g, 0] == g)
    def _():
        xs = xs_ref[...]
        gu = lax.dot_general(xs, gup_ref[0], (((1,), (1,)), ((), ())),
                             preferred_element_type=jnp.float32)
        gate = gu[:, :DFF]
        up = gu[:, DFF:]
        h = gate * jax.nn.sigmoid(gate) * up
        y_ref[...] = lax.dot_general(h, down_ref[0], (((1,), (1,)), ((), ())),
                                     preferred_element_type=jnp.float32)


def _ragged_gemm(bmap, bexp, xs, gate_up_proj, down_proj):
    grid_spec = pltpu.PrefetchScalarGridSpec(
        num_scalar_prefetch=2,
        grid=(G,),
        in_specs=[
            pl.BlockSpec((BT, D), lambda g, bm, be: (bm[g, 0], 0)),
            pl.BlockSpec((1, 2 * DFF, D), lambda g, bm, be: (be[g, 0], 0, 0)),
            pl.BlockSpec((1, D, DFF), lambda g, bm, be: (be[g, 0], 0, 0)),
        ],
        out_specs=pl.BlockSpec((BT, D), lambda g, bm, be: (bm[g, 0], 0)),
    )
    return pl.pallas_call(
        _gemm_body,
        grid_spec=grid_spec,
        out_shape=jax.ShapeDtypeStruct((P_ROWS, D), jnp.float32),
    )(bmap, bexp, xs, gate_up_proj, down_proj)


_TSUB = 16


_N_SUB = CHUNK // _TSUB


def _sc_combine_body(y_hbm, dmat_hbm, ws_hbm, out_hbm,
                     i0a, i0b, i1a, i1b, w0a, w0b, w1a, w1b,
                     aa, ab, ba, bb, oa, ob,
                     gsa, gsb, osa, osb):
    wid = lax.axis_index("s") * _NC + lax.axis_index("c")
    i0 = (i0a, i0b)
    i1 = (i1a, i1b)
    w0 = (w0a, w0b)
    w1 = (w1a, w1b)
    av = (aa, ab)
    bv = (ba, bb)
    ov = (oa, ob)
    gsem = (gsa, gsb)
    osem = (osa, osb)

    def issue(s, bi):
        col = s * _TSUB
        row0 = wid * CHUNK + col
        pltpu.sync_copy(dmat_hbm.at[wid, pl.ds(col, _TSUB)], i0[bi])
        pltpu.sync_copy(dmat_hbm.at[wid + _NW, pl.ds(col, _TSUB)], i1[bi])
        pltpu.sync_copy(ws_hbm.at[pl.ds(row0, _TSUB)], w0[bi])
        pltpu.sync_copy(ws_hbm.at[pl.ds(N_TOK + row0, _TSUB)], w1[bi])
        ca = pltpu.async_copy(y_hbm.at[i0[bi]], av[bi], gsem[bi])
        cb = pltpu.async_copy(y_hbm.at[i1[bi]], bv[bi], gsem[bi])
        return ca, cb

    pend = issue(0, 0)
    out_pend = [None, None]
    for s in range(_N_SUB):
        bi = s % 2
        pend[0].wait()
        pend[1].wait()
        if s + 1 < _N_SUB:
            pend = issue(s + 1, 1 - bi)
        if out_pend[bi] is not None:
            out_pend[bi].wait()

        def row(r, _, bi=bi):
            s0 = w0[bi][r, :]
            s1 = w1[bi][r, :]

            @plsc.parallel_loop(0, D, step=16, unroll=8)
            def col(jj):
                ov[bi][r, pl.ds(jj, 16)] = (
                    s0 * av[bi][r, pl.ds(jj, 16)]
                    + s1 * bv[bi][r, pl.ds(jj, 16)])

            return 0

        lax.fori_loop(0, _TSUB, row, 0)
        row0 = wid * CHUNK + s * _TSUB
        out_pend[bi] = pltpu.async_copy(ov[bi], out_hbm.at[pl.ds(row0, _TSUB)],
                                        osem[bi])
    for cp in out_pend:
        if cp is not None:
            cp.wait()


def _sc_combine(y, d_mat, w_splat):
    mesh = plsc.VectorSubcoreMesh(core_axis_name="c", subcore_axis_name="s", num_cores=_NC, num_subcores=_NS)
    f = pl.kernel(
        _sc_combine_body,
        out_type=jax.ShapeDtypeStruct((N_TOK, D), jnp.float32),
        mesh=mesh,
        scratch_types=(
            [pltpu.VMEM((_TSUB,), jnp.int32)] * 4
            + [pltpu.VMEM((_TSUB, 16), jnp.float32)] * 4
            + [pltpu.VMEM((_TSUB, D), jnp.float32)] * 6
            + [pltpu.SemaphoreType.DMA] * 4
        ),
    )
    return f(y, d_mat, w_splat)


def kernel(hidden_states, gate_weight, gate_up_proj, down_proj, similarity_matrix):
    B, S, Dm = hidden_states.shape
    x = hidden_states.reshape(-1, Dm)
    tri = jnp.tril(jnp.ones((256, 256), jnp.float32))
    d, w_splat, bmap, bexp = _routing(x, gate_weight, similarity_matrix, tri)
    d_mat = d.reshape(N_CHUNK, CHUNK)
    xs = _sc_scatter(x, d_mat)
    y = _ragged_gemm(bmap, bexp, xs, gate_up_proj, down_proj)
    out = _sc_combine(y, d_mat, w_splat)
    return out.reshape(B, S, Dm)


# speedup vs baseline: 1.3552x; 1.0973x over previous
import functools

import jax
import jax.numpy as jnp
from jax import lax
from jax.experimental import pallas as pl
from jax.experimental.pallas import tpu as pltpu
from jax.experimental.pallas import tpu_sc as plsc

N_EXP = 8
D = 1024
DFF = 512
N_TOK = 2048
N_PAIR = 2 * N_TOK
BT = 512
G = N_PAIR // BT + N_EXP
P_ROWS = G * BT
CHUNK = 64
N_CHUNK = N_PAIR // CHUNK
NEG = -3.0e38


def _argmax_lanes(v, iota_row):
    m = jnp.max(v, axis=-1, keepdims=True)
    return jnp.min(jnp.where(v == m, iota_row, N_EXP), axis=-1, keepdims=True), m


def _routing_body(x_ref, gw_ref, sim_ref, tri_ref, d_ref, w_ref, bmap_ref,
                  bexp_ref):
    x = x_ref[...]
    gw = gw_ref[...]
    logits = lax.dot_general(x, gw, (((1,), (1,)), ((), ())),
                             preferred_element_type=jnp.float32)
    m = jnp.max(logits, axis=-1, keepdims=True)
    e = jnp.exp(logits - m)
    probs = e / jnp.sum(e, axis=-1, keepdims=True)

    iota_row = lax.broadcasted_iota(jnp.int32, (N_TOK, N_EXP), 1)
    i1, v1 = _argmax_lanes(probs, iota_row)
    oh1 = (iota_row == i1)
    probs2 = jnp.where(oh1, NEG, probs)
    i2, v2 = _argmax_lanes(probs2, iota_row)
    oh2 = (iota_row == i2)
    denom = jnp.maximum(v1 + v2, 1e-12)
    w1 = v1 / denom
    w2 = v2 / denom

    mask_col = jnp.max(oh1.astype(jnp.float32), axis=0, keepdims=True)
    sim = sim_ref[...]
    iota_r8 = lax.broadcasted_iota(jnp.int32, (N_EXP, N_EXP), 1)
    iota_c8 = lax.broadcasted_iota(jnp.int32, (N_EXP, N_EXP), 0)
    eye = (iota_r8 == iota_c8)
    sim_masked = jnp.where(mask_col > 0.5, sim, NEG)
    best_sim = jnp.max(sim_masked, axis=-1, keepdims=True)
    best_j = jnp.min(jnp.where(sim_masked == best_sim, iota_r8, N_EXP),
                     axis=-1, keepdims=True)
    mask_row = jnp.max(jnp.where(eye, jnp.broadcast_to(mask_col, (N_EXP, N_EXP)),
                                 0.0), axis=-1, keepdims=True)
    reroute = (mask_row < 0.5) & (best_sim >= 0.5)
    ident = lax.broadcasted_iota(jnp.int32, (N_EXP, 1), 0)
    emap = jnp.where(reroute, best_j, ident)
    perm = (emap == iota_r8).astype(jnp.float32)

    pre = w1 * oh1.astype(jnp.float32) + w2 * oh2.astype(jnp.float32)
    rw = lax.dot_general(pre, perm, (((1,), (0,)), ((), ())),
                         preferred_element_type=jnp.float32)

    f1, u1 = _argmax_lanes(rw, iota_row)
    ohf1 = (iota_row == f1)
    rwm = jnp.where(ohf1, -1.0, rw)
    f2, u2 = _argmax_lanes(rwm, iota_row)
    ohf2 = (iota_row == f2)

    w_ref[:N_TOK, :] = jnp.broadcast_to(u1, (N_TOK, 16))
    w_ref[N_TOK:, :] = jnp.broadcast_to(u2, (N_TOK, 16))

    TB = 256
    n_blk = N_PAIR // TB
    tri = tri_ref[...]
    o_blks, c_blks, carries = [], [], []
    car = jnp.zeros((1, N_EXP), jnp.float32)
    for b in range(n_blk):
        if b < n_blk // 2:
            o_blk = ohf1[b * TB:(b + 1) * TB, :].astype(jnp.float32)
        else:
            o_blk = ohf2[(b - n_blk // 2) * TB:(b - n_blk // 2 + 1) * TB,
                         :].astype(jnp.float32)
        c = lax.dot_general(tri, o_blk, (((1,), (0,)), ((), ())),
                            preferred_element_type=jnp.float32)
        o_blks.append(o_blk)
        c_blks.append(c)
        carries.append(car)
        car = car + c[TB - 1:TB, :]

    counts = car
    nb = jnp.floor((counts + float(BT - 1)) * (1.0 / BT))
    iota_u8 = lax.broadcasted_iota(jnp.int32, (N_EXP, N_EXP), 0)
    u8 = (iota_u8 < iota_r8).astype(jnp.float32)
    excl = lax.dot_general(nb, u8, (((1,), (0,)), ((), ())),
                           preferred_element_type=jnp.float32)
    pad_base = excl * float(BT)
    total = jnp.sum(nb)
    cumnext = excl + nb

    for b in range(n_blk):
        inc = c_blks[b] + carries[b]
        rank = jnp.sum(inc * o_blks[b], axis=-1, keepdims=True) - 1.0
        pb = jnp.sum(pad_base * o_blks[b], axis=-1, keepdims=True)
        d_ref[b * TB:(b + 1) * TB, :] = (pb + rank).astype(jnp.int32)

    gcol = lax.broadcasted_iota(jnp.int32, (G, 1), 0).astype(jnp.float32)
    bmapf = jnp.minimum(gcol, total - 1.0)
    bexp = jnp.sum((jnp.broadcast_to(cumnext, (G, N_EXP)) <= bmapf)
                   .astype(jnp.int32), axis=-1, keepdims=True)
    bmap_ref[...] = bmapf.astype(jnp.int32)
    bexp_ref[...] = bexp


def _routing(x, gate_weight, sim, tri):
    return pl.pallas_call(
        _routing_body,
        out_shape=(
            jax.ShapeDtypeStruct((N_PAIR, 1), jnp.int32),
            jax.ShapeDtypeStruct((N_PAIR, 16), jnp.float32),
            jax.ShapeDtypeStruct((G, 1), jnp.int32),
            jax.ShapeDtypeStruct((G, 1), jnp.int32),
        ),
        in_specs=[
            pl.BlockSpec((N_TOK, D), lambda: (0, 0)),
            pl.BlockSpec((N_EXP, D), lambda: (0, 0)),
            pl.BlockSpec((N_EXP, N_EXP), lambda: (0, 0)),
            pl.BlockSpec((256, 256), lambda: (0, 0)),
        ],
        out_specs=(
            pl.BlockSpec((N_PAIR, 1), lambda: (0, 0)),
            pl.BlockSpec((N_PAIR, 16), lambda: (0, 0)),
            pl.BlockSpec((G, 1), lambda: (0, 0)),
            pl.BlockSpec((G, 1), lambda: (0, 0)),
        ),
    )(x, gate_weight, sim, tri)


_NC = 2
_NS = 16
_NW = _NC * _NS


def _sc_scatter_body(x_hbm, dmat_hbm, xs_hbm, idx0_v, idx1_v, rows0_v,
                     rsem, ssem):
    wid = lax.axis_index("s") * _NC + lax.axis_index("c")
    t0 = wid * CHUNK
    pltpu.sync_copy(dmat_hbm.at[wid], idx0_v)
    pltpu.sync_copy(dmat_hbm.at[wid + _NW], idx1_v)
    rows_cp = pltpu.async_copy(x_hbm.at[pl.ds(t0, CHUNK)], rows0_v, rsem)
    rows_cp.wait()
    sc0 = pltpu.async_copy(rows0_v, xs_hbm.at[idx0_v], ssem)
    sc1 = pltpu.async_copy(rows0_v, xs_hbm.at[idx1_v], ssem)
    sc0.wait()
    sc1.wait()


def _sc_scatter(x, d_mat):
    mesh = plsc.VectorSubcoreMesh(core_axis_name="c", subcore_axis_name="s", num_cores=_NC, num_subcores=_NS)
    f = pl.kernel(
        _sc_scatter_body,
        out_type=jax.ShapeDtypeStruct((P_ROWS, D), jnp.float32),
        mesh=mesh,
        scratch_types=[
            pltpu.VMEM((CHUNK,), jnp.int32),
            pltpu.VMEM((CHUNK,), jnp.int32),
            pltpu.VMEM((CHUNK, D), jnp.float32),
            pltpu.SemaphoreType.DMA,
            pltpu.SemaphoreType.DMA,
        ],
    )
    return f(x, d_mat)


def _gemm_body(bmap_ref, bexp_ref, xs_ref, gup_ref, down_ref, y_ref):
    g = pl.program_id(0)

    @pl.when(bmap_ref[g, 0] == g)
    def _():
        xs = xs_ref[...]
        gu = lax.dot_general(xs, gup_ref[0], (((1,), (1,)), ((), ())),
                             preferred_element_type=jnp.float32)
        gate = gu[:, :DFF]
        up = gu[:, DFF:]
        h = gate * jax.nn.sigmoid(gate) * up
        y_ref[...] = lax.dot_general(h, down_ref[0], (((1,), (1,)), ((), ())),
                                     preferred_element_type=jnp.float32)


def _ragged_gemm(bmap, bexp, xs, gate_up_proj, down_proj):
    grid_spec = pltpu.PrefetchScalarGridSpec(
        num_scalar_prefetch=2,
        grid=(G,),
        in_specs=[
            pl.BlockSpec((BT, D), lambda g, bm, be: (bm[g, 0], 0)),
            pl.BlockSpec((1, 2 * DFF, D), lambda g, bm, be: (be[g, 0], 0, 0)),
            pl.BlockSpec((1, D, DFF), lambda g, bm, be: (be[g, 0], 0, 0)),
        ],
        out_specs=pl.BlockSpec((BT, D), lambda g, bm, be: (bm[g, 0], 0)),
    )
    return pl.pallas_call(
        _gemm_body,
        grid_spec=grid_spec,
        out_shape=jax.ShapeDtypeStruct((P_ROWS, D), jnp.float32),
    )(bmap, bexp, xs, gate_up_proj, down_proj)


_TSUB = 16


_N_SUB = CHUNK // _TSUB


def _sc_combine_body(y_hbm, dmat_hbm, ws_hbm, out_hbm,
                     i0a, i0b, i1a, i1b, w0a, w0b, w1a, w1b,
                     aa, ab, ba, bb, oa, ob,
                     gsa, gsb, osa, osb):
    wid = lax.axis_index("s") * _NC + lax.axis_index("c")
    i0 = (i0a, i0b)
    i1 = (i1a, i1b)
    w0 = (w0a, w0b)
    w1 = (w1a, w1b)
    av = (aa, ab)
    bv = (ba, bb)
    ov = (oa, ob)
    gsem = (gsa, gsb)
    osem = (osa, osb)

    def issue(s, bi):
        col = s * _TSUB
        row0 = wid * CHUNK + col
        pltpu.sync_copy(dmat_hbm.at[wid, pl.ds(col, _TSUB)], i0[bi])
        pltpu.sync_copy(dmat_hbm.at[wid + _NW, pl.ds(col, _TSUB)], i1[bi])
        pltpu.sync_copy(ws_hbm.at[pl.ds(row0, _TSUB)], w0[bi])
        pltpu.sync_copy(ws_hbm.at[pl.ds(N_TOK + row0, _TSUB)], w1[bi])
        ca = pltpu.async_copy(y_hbm.at[i0[bi]], av[bi], gsem[bi])
        cb = pltpu.async_copy(y_hbm.at[i1[bi]], bv[bi], gsem[bi])
        return ca, cb

    pend = issue(0, 0)
    out_pend = [None, None]
    for s in range(_N_SUB):
        bi = s % 2
        pend[0].wait()
        pend[1].wait()
        if s + 1 < _N_SUB:
            pend = issue(s + 1, 1 - bi)
        if out_pend[bi] is not None:
            out_pend[bi].wait()

        def row(r, _, bi=bi):
            s0 = w0[bi][r, :]
            s1 = w1[bi][r, :]

            @plsc.parallel_loop(0, D, step=16, unroll=8)
            def col(jj):
                ov[bi][r, pl.ds(jj, 16)] = (
                    s0 * av[bi][r, pl.ds(jj, 16)]
                    + s1 * bv[bi][r, pl.ds(jj, 16)])

            return 0

        lax.fori_loop(0, _TSUB, row, 0)
        row0 = wid * CHUNK + s * _TSUB
        out_pend[bi] = pltpu.async_copy(ov[bi], out_hbm.at[pl.ds(row0, _TSUB)],
                                        osem[bi])
    for cp in out_pend:
        if cp is not None:
            cp.wait()


def _sc_combine(y, d_mat, w_splat):
    mesh = plsc.VectorSubcoreMesh(core_axis_name="c", subcore_axis_name="s", num_cores=_NC, num_subcores=_NS)
    f = pl.kernel(
        _sc_combine_body,
        out_type=jax.ShapeDtypeStruct((N_TOK, D), jnp.float32),
        mesh=mesh,
        scratch_types=(
            [pltpu.VMEM((_TSUB,), jnp.int32)] * 4
            + [pltpu.VMEM((_TSUB, 16), jnp.float32)] * 4
            + [pltpu.VMEM((_TSUB, D), jnp.float32)] * 6
            + [pltpu.SemaphoreType.DMA] * 4
        ),
    )
    return f(y, d_mat, w_splat)


def kernel(hidden_states, gate_weight, gate_up_proj, down_proj, similarity_matrix):
    B, S, Dm = hidden_states.shape
    x = hidden_states.reshape(-1, Dm)
    tri = jnp.tril(jnp.ones((256, 256), jnp.float32))
    d, w_splat, bmap, bexp = _routing(x, gate_weight, similarity_matrix, tri)
    d_mat = d.reshape(N_CHUNK, CHUNK)
    xs = _sc_scatter(x, d_mat)
    y = _ragged_gemm(bmap, bexp, xs, gate_up_proj, down_proj)
    out = _sc_combine(y, d_mat, w_splat)
    return out.reshape(B, S, Dm)


# speedup vs baseline: 1.4517x; 1.0712x over previous
import functools

import jax
import jax.numpy as jnp
from jax import lax
from jax.experimental import pallas as pl
from jax.experimental.pallas import tpu as pltpu
from jax.experimental.pallas import tpu_sc as plsc

N_EXP = 8
D = 1024
DFF = 512
N_TOK = 2048
N_PAIR = 2 * N_TOK
BT = 512
G = N_PAIR // BT + N_EXP
P_ROWS = G * BT
CHUNK = 64
N_CHUNK = N_PAIR // CHUNK
NEG = -3.0e38


def _argmax_lanes(v, iota_row):
    m = jnp.max(v, axis=-1, keepdims=True)
    return jnp.min(jnp.where(v == m, iota_row, N_EXP), axis=-1, keepdims=True), m


def _routing_body(x_ref, gw_ref, sim_ref, tri_ref, d_ref, w_ref, bmap_ref,
                  bexp_ref):
    x = x_ref[...]
    gw = gw_ref[...]
    logits = lax.dot_general(x, gw, (((1,), (1,)), ((), ())),
                             preferred_element_type=jnp.float32)
    m = jnp.max(logits, axis=-1, keepdims=True)
    e = jnp.exp(logits - m)
    probs = e / jnp.sum(e, axis=-1, keepdims=True)

    iota_row = lax.broadcasted_iota(jnp.int32, (N_TOK, N_EXP), 1)
    i1, v1 = _argmax_lanes(probs, iota_row)
    oh1 = (iota_row == i1)
    probs2 = jnp.where(oh1, NEG, probs)
    i2, v2 = _argmax_lanes(probs2, iota_row)
    oh2 = (iota_row == i2)
    denom = jnp.maximum(v1 + v2, 1e-12)
    w1 = v1 / denom
    w2 = v2 / denom

    mask_col = jnp.max(oh1.astype(jnp.float32), axis=0, keepdims=True)
    sim = sim_ref[...]
    iota_r8 = lax.broadcasted_iota(jnp.int32, (N_EXP, N_EXP), 1)
    iota_c8 = lax.broadcasted_iota(jnp.int32, (N_EXP, N_EXP), 0)
    eye = (iota_r8 == iota_c8)
    sim_masked = jnp.where(mask_col > 0.5, sim, NEG)
    best_sim = jnp.max(sim_masked, axis=-1, keepdims=True)
    best_j = jnp.min(jnp.where(sim_masked == best_sim, iota_r8, N_EXP),
                     axis=-1, keepdims=True)
    mask_row = jnp.max(jnp.where(eye, jnp.broadcast_to(mask_col, (N_EXP, N_EXP)),
                                 0.0), axis=-1, keepdims=True)
    reroute = (mask_row < 0.5) & (best_sim >= 0.5)
    ident = lax.broadcasted_iota(jnp.int32, (N_EXP, 1), 0)
    emap = jnp.where(reroute, best_j, ident)
    perm = (emap == iota_r8).astype(jnp.float32)

    pre = w1 * oh1.astype(jnp.float32) + w2 * oh2.astype(jnp.float32)
    rw = lax.dot_general(pre, perm, (((1,), (0,)), ((), ())),
                         preferred_element_type=jnp.float32)

    f1, u1 = _argmax_lanes(rw, iota_row)
    ohf1 = (iota_row == f1)
    rwm = jnp.where(ohf1, -1.0, rw)
    f2, u2 = _argmax_lanes(rwm, iota_row)
    ohf2 = (iota_row == f2)

    w_ref[:N_TOK, :] = jnp.broadcast_to(u1, (N_TOK, 16))
    w_ref[N_TOK:, :] = jnp.broadcast_to(u2, (N_TOK, 16))

    TB = 256
    n_blk = N_PAIR // TB
    tri = tri_ref[...]
    o_blks, c_blks, carries = [], [], []
    car = jnp.zeros((1, N_EXP), jnp.float32)
    for b in range(n_blk):
        if b < n_blk // 2:
            o_blk = ohf1[b * TB:(b + 1) * TB, :].astype(jnp.float32)
        else:
            o_blk = ohf2[(b - n_blk // 2) * TB:(b - n_blk // 2 + 1) * TB,
                         :].astype(jnp.float32)
        c = lax.dot_general(tri, o_blk, (((1,), (0,)), ((), ())),
                            preferred_element_type=jnp.float32)
        o_blks.append(o_blk)
        c_blks.append(c)
        carries.append(car)
        car = car + c[TB - 1:TB, :]

    counts = car
    nb = jnp.floor((counts + float(BT - 1)) * (1.0 / BT))
    iota_u8 = lax.broadcasted_iota(jnp.int32, (N_EXP, N_EXP), 0)
    u8 = (iota_u8 < iota_r8).astype(jnp.float32)
    excl = lax.dot_general(nb, u8, (((1,), (0,)), ((), ())),
                           preferred_element_type=jnp.float32)
    pad_base = excl * float(BT)
    total = jnp.sum(nb)
    cumnext = excl + nb

    for b in range(n_blk):
        inc = c_blks[b] + carries[b]
        rank = jnp.sum(inc * o_blks[b], axis=-1, keepdims=True) - 1.0
        pb = jnp.sum(pad_base * o_blks[b], axis=-1, keepdims=True)
        d_ref[b * TB:(b + 1) * TB, :] = (pb + rank).astype(jnp.int32)

    gcol = lax.broadcasted_iota(jnp.int32, (G, 1), 0).astype(jnp.float32)
    bmapf = jnp.minimum(gcol, total - 1.0)
    bexp = jnp.sum((jnp.broadcast_to(cumnext, (G, N_EXP)) <= bmapf)
                   .astype(jnp.int32), axis=-1, keepdims=True)
    bmap_ref[...] = bmapf.astype(jnp.int32)
    bexp_ref[...] = bexp


def _routing(x, gate_weight, sim, tri):
    return pl.pallas_call(
        _routing_body,
        out_shape=(
            jax.ShapeDtypeStruct((N_PAIR, 1), jnp.int32),
            jax.ShapeDtypeStruct((N_PAIR, 16), jnp.float32),
            jax.ShapeDtypeStruct((G, 1), jnp.int32),
            jax.ShapeDtypeStruct((G, 1), jnp.int32),
        ),
        in_specs=[
            pl.BlockSpec((N_TOK, D), lambda: (0, 0)),
            pl.BlockSpec((N_EXP, D), lambda: (0, 0)),
            pl.BlockSpec((N_EXP, N_EXP), lambda: (0, 0)),
            pl.BlockSpec((256, 256), lambda: (0, 0)),
        ],
        out_specs=(
            pl.BlockSpec((N_PAIR, 1), lambda: (0, 0)),
            pl.BlockSpec((N_PAIR, 16), lambda: (0, 0)),
            pl.BlockSpec((G, 1), lambda: (0, 0)),
            pl.BlockSpec((G, 1), lambda: (0, 0)),
        ),
    )(x, gate_weight, sim, tri)


_NC = 2
_NS = 16
_NW = _NC * _NS


def _sc_scatter_body(x_hbm, dmat_hbm, xs_hbm, idx0_v, idx1_v, rows0_v,
                     rsem, ssem):
    wid = lax.axis_index("s") * _NC + lax.axis_index("c")
    t0 = wid * CHUNK
    pltpu.sync_copy(dmat_hbm.at[wid], idx0_v)
    pltpu.sync_copy(dmat_hbm.at[wid + _NW], idx1_v)
    rows_cp = pltpu.async_copy(x_hbm.at[pl.ds(t0, CHUNK)], rows0_v, rsem)
    rows_cp.wait()
    sc0 = pltpu.async_copy(rows0_v, xs_hbm.at[idx0_v], ssem)
    sc1 = pltpu.async_copy(rows0_v, xs_hbm.at[idx1_v], ssem)
    sc0.wait()
    sc1.wait()


def _sc_scatter(x, d_mat):
    mesh = plsc.VectorSubcoreMesh(core_axis_name="c", subcore_axis_name="s", num_cores=_NC, num_subcores=_NS)
    f = pl.kernel(
        _sc_scatter_body,
        out_type=jax.ShapeDtypeStruct((P_ROWS, D), jnp.float32),
        mesh=mesh,
        scratch_types=[
            pltpu.VMEM((CHUNK,), jnp.int32),
            pltpu.VMEM((CHUNK,), jnp.int32),
            pltpu.VMEM((CHUNK, D), jnp.float32),
            pltpu.SemaphoreType.DMA,
            pltpu.SemaphoreType.DMA,
        ],
    )
    return f(x, d_mat)


def _gemm_body(bmap_ref, bexp_ref, xs_ref, gup_ref, down_ref, y_ref):
    g = pl.program_id(0)

    @pl.when(bmap_ref[g, 0] == g)
    def _():
        xs = xs_ref[...]
        gu = lax.dot_general(xs, gup_ref[0], (((1,), (1,)), ((), ())),
                             preferred_element_type=jnp.float32)
        gate = gu[:, :DFF]
        up = gu[:, DFF:]
        h = gate * jax.nn.sigmoid(gate) * up
        y_ref[...] = lax.dot_general(h, down_ref[0], (((1,), (1,)), ((), ())),
                                     preferred_element_type=jnp.float32)


def _ragged_gemm(bmap, bexp, xs, gate_up_proj, down_proj):
    grid_spec = pltpu.PrefetchScalarGridSpec(
        num_scalar_prefetch=2,
        grid=(G,),
        in_specs=[
            pl.BlockSpec((BT, D), lambda g, bm, be: (bm[g, 0], 0)),
            pl.BlockSpec((1, 2 * DFF, D), lambda g, bm, be: (be[g, 0], 0, 0)),
            pl.BlockSpec((1, D, DFF), lambda g, bm, be: (be[g, 0], 0, 0)),
        ],
        out_specs=pl.BlockSpec((BT, D), lambda g, bm, be: (bm[g, 0], 0)),
    )
    return pl.pallas_call(
        _gemm_body,
        grid_spec=grid_spec,
        out_shape=jax.ShapeDtypeStruct((P_ROWS, D), jnp.float32),
    )(bmap, bexp, xs, gate_up_proj, down_proj)


_TSUB = 16


_N_SUB = CHUNK // _TSUB


def _sc_combine_body(y_hbm, dmat_hbm, ws_hbm, out_hbm,
                     i0all, i1all, w0all, w1all,
                     aa, ab, ba, bb, oa, ob,
                     lsem, gsa, gsb, osa, osb):
    wid = lax.axis_index("s") * _NC + lax.axis_index("c")
    av = (aa, ab)
    bv = (ba, bb)
    ov = (oa, ob)
    gsem = (gsa, gsb)
    osem = (osa, osb)

    row_base = wid * CHUNK
    hoist = (
        pltpu.async_copy(dmat_hbm.at[wid], i0all, lsem),
        pltpu.async_copy(dmat_hbm.at[wid + _NW], i1all, lsem),
        pltpu.async_copy(ws_hbm.at[pl.ds(row_base, CHUNK)], w0all, lsem),
        pltpu.async_copy(ws_hbm.at[pl.ds(N_TOK + row_base, CHUNK)], w1all,
                         lsem),
    )
    for cp in hoist:
        cp.wait()

    def issue(s, bi):
        col = s * _TSUB
        ca = pltpu.async_copy(y_hbm.at[i0all[pl.ds(col, _TSUB)]], av[bi],
                              gsem[bi])
        cb = pltpu.async_copy(y_hbm.at[i1all[pl.ds(col, _TSUB)]], bv[bi],
                              gsem[bi])
        return ca, cb

    pend = issue(0, 0)
    out_pend = [None, None]
    for s in range(_N_SUB):
        bi = s % 2
        col = s * _TSUB
        pend[0].wait()
        pend[1].wait()
        if s + 1 < _N_SUB:
            pend = issue(s + 1, 1 - bi)
        if out_pend[bi] is not None:
            out_pend[bi].wait()

        def row(r, _, bi=bi, col=col):
            s0 = w0all[col + r, :]
            s1 = w1all[col + r, :]

            @plsc.parallel_loop(0, D, step=16, unroll=8)
            def _(jj):
                ov[bi][r, pl.ds(jj, 16)] = (
                    s0 * av[bi][r, pl.ds(jj, 16)]
                    + s1 * bv[bi][r, pl.ds(jj, 16)])

            return 0

        lax.fori_loop(0, _TSUB, row, 0)
        out_pend[bi] = pltpu.async_copy(
            ov[bi], out_hbm.at[pl.ds(row_base + col, _TSUB)], osem[bi])
    for cp in out_pend:
        if cp is not None:
            cp.wait()


def _sc_combine(y, d_mat, w_splat):
    mesh = plsc.VectorSubcoreMesh(core_axis_name="c", subcore_axis_name="s", num_cores=_NC, num_subcores=_NS)
    f = pl.kernel(
        _sc_combine_body,
        out_type=jax.ShapeDtypeStruct((N_TOK, D), jnp.float32),
        mesh=mesh,
        scratch_types=(
            [pltpu.VMEM((CHUNK,), jnp.int32)] * 2
            + [pltpu.VMEM((CHUNK, 16), jnp.float32)] * 2
            + [pltpu.VMEM((_TSUB, D), jnp.float32)] * 6
            + [pltpu.SemaphoreType.DMA] * 5
        ),
    )
    return f(y, d_mat, w_splat)


def kernel(hidden_states, gate_weight, gate_up_proj, down_proj, similarity_matrix):
    B, S, Dm = hidden_states.shape
    x = hidden_states.reshape(-1, Dm)
    tri = jnp.tril(jnp.ones((256, 256), jnp.float32))
    d, w_splat, bmap, bexp = _routing(x, gate_weight, similarity_matrix, tri)
    d_mat = d.reshape(N_CHUNK, CHUNK)
    xs = _sc_scatter(x, d_mat)
    y = _ragged_gemm(bmap, bexp, xs, gate_up_proj, down_proj)
    out = _sc_combine(y, d_mat, w_splat)
    return out.reshape(B, S, Dm)


# speedup vs baseline: 1.4634x; 1.0080x over previous
import functools

import jax
import jax.numpy as jnp
from jax import lax
from jax.experimental import pallas as pl
from jax.experimental.pallas import tpu as pltpu
from jax.experimental.pallas import tpu_sc as plsc

N_EXP = 8
D = 1024
DFF = 512
N_TOK = 2048
N_PAIR = 2 * N_TOK
BT = 512
G = N_PAIR // BT + N_EXP
P_ROWS = G * BT
CHUNK = 64
N_CHUNK = N_PAIR // CHUNK
NEG = -3.0e38


def _argmax_lanes(v, iota_row):
    m = jnp.max(v, axis=-1, keepdims=True)
    return jnp.min(jnp.where(v == m, iota_row, N_EXP), axis=-1, keepdims=True), m


def _routing_body(x_ref, gw_ref, sim_ref, tri_ref, d_ref, w_ref, bmap_ref,
                  bexp_ref):
    x = x_ref[...]
    gw = gw_ref[...]
    logits = lax.dot_general(x, gw, (((1,), (1,)), ((), ())),
                             preferred_element_type=jnp.float32)
    m = jnp.max(logits, axis=-1, keepdims=True)
    e = jnp.exp(logits - m)
    probs = e / jnp.sum(e, axis=-1, keepdims=True)

    iota_row = lax.broadcasted_iota(jnp.int32, (N_TOK, N_EXP), 1)
    i1, v1 = _argmax_lanes(probs, iota_row)
    oh1 = (iota_row == i1)
    probs2 = jnp.where(oh1, NEG, probs)
    i2, v2 = _argmax_lanes(probs2, iota_row)
    oh2 = (iota_row == i2)
    denom = jnp.maximum(v1 + v2, 1e-12)
    w1 = v1 / denom
    w2 = v2 / denom

    mask_col = jnp.max(oh1.astype(jnp.float32), axis=0, keepdims=True)
    sim = sim_ref[...]
    iota_r8 = lax.broadcasted_iota(jnp.int32, (N_EXP, N_EXP), 1)
    iota_c8 = lax.broadcasted_iota(jnp.int32, (N_EXP, N_EXP), 0)
    eye = (iota_r8 == iota_c8)
    sim_masked = jnp.where(mask_col > 0.5, sim, NEG)
    best_sim = jnp.max(sim_masked, axis=-1, keepdims=True)
    best_j = jnp.min(jnp.where(sim_masked == best_sim, iota_r8, N_EXP),
                     axis=-1, keepdims=True)
    mask_row = jnp.max(jnp.where(eye, jnp.broadcast_to(mask_col, (N_EXP, N_EXP)),
                                 0.0), axis=-1, keepdims=True)
    reroute = (mask_row < 0.5) & (best_sim >= 0.5)
    ident = lax.broadcasted_iota(jnp.int32, (N_EXP, 1), 0)
    emap = jnp.where(reroute, best_j, ident)
    perm = (emap == iota_r8).astype(jnp.float32)

    pre = w1 * oh1.astype(jnp.float32) + w2 * oh2.astype(jnp.float32)
    rw = lax.dot_general(pre, perm, (((1,), (0,)), ((), ())),
                         preferred_element_type=jnp.float32)

    f1, u1 = _argmax_lanes(rw, iota_row)
    ohf1 = (iota_row == f1)
    rwm = jnp.where(ohf1, -1.0, rw)
    f2, u2 = _argmax_lanes(rwm, iota_row)
    ohf2 = (iota_row == f2)

    w_ref[:N_TOK, :] = jnp.broadcast_to(u1, (N_TOK, 16))
    w_ref[N_TOK:, :] = jnp.broadcast_to(u2, (N_TOK, 16))

    TB = 256
    n_blk = N_PAIR // TB
    tri = tri_ref[...]
    o_blks, c_blks, carries = [], [], []
    car = jnp.zeros((1, N_EXP), jnp.float32)
    for b in range(n_blk):
        if b < n_blk // 2:
            o_blk = ohf1[b * TB:(b + 1) * TB, :].astype(jnp.float32)
        else:
            o_blk = ohf2[(b - n_blk // 2) * TB:(b - n_blk // 2 + 1) * TB,
                         :].astype(jnp.float32)
        c = lax.dot_general(tri, o_blk, (((1,), (0,)), ((), ())),
                            preferred_element_type=jnp.float32)
        o_blks.append(o_blk)
        c_blks.append(c)
        carries.append(car)
        car = car + c[TB - 1:TB, :]

    counts = car
    nb = jnp.floor((counts + float(BT - 1)) * (1.0 / BT))
    iota_u8 = lax.broadcasted_iota(jnp.int32, (N_EXP, N_EXP), 0)
    u8 = (iota_u8 < iota_r8).astype(jnp.float32)
    excl = lax.dot_general(nb, u8, (((1,), (0,)), ((), ())),
                           preferred_element_type=jnp.float32)
    pad_base = excl * float(BT)
    total = jnp.sum(nb)
    cumnext = excl + nb

    for b in range(n_blk):
        inc = c_blks[b] + carries[b]
        rank = jnp.sum(inc * o_blks[b], axis=-1, keepdims=True) - 1.0
        pb = jnp.sum(pad_base * o_blks[b], axis=-1, keepdims=True)
        d_ref[b * TB:(b + 1) * TB, :] = (pb + rank).astype(jnp.int32)

    gcol = lax.broadcasted_iota(jnp.int32, (G, 1), 0).astype(jnp.float32)
    bmapf = jnp.minimum(gcol, total - 1.0)
    bexp = jnp.sum((jnp.broadcast_to(cumnext, (G, N_EXP)) <= bmapf)
                   .astype(jnp.int32), axis=-1, keepdims=True)
    bmap_ref[...] = bmapf.astype(jnp.int32)
    bexp_ref[...] = bexp


def _routing(x, gate_weight, sim, tri):
    return pl.pallas_call(
        _routing_body,
        out_shape=(
            jax.ShapeDtypeStruct((N_PAIR, 1), jnp.int32),
            jax.ShapeDtypeStruct((N_PAIR, 16), jnp.float32),
            jax.ShapeDtypeStruct((G, 1), jnp.int32),
            jax.ShapeDtypeStruct((G, 1), jnp.int32),
        ),
        in_specs=[
            pl.BlockSpec((N_TOK, D), lambda: (0, 0)),
            pl.BlockSpec((N_EXP, D), lambda: (0, 0)),
            pl.BlockSpec((N_EXP, N_EXP), lambda: (0, 0)),
            pl.BlockSpec((256, 256), lambda: (0, 0)),
        ],
        out_specs=(
            pl.BlockSpec((N_PAIR, 1), lambda: (0, 0)),
            pl.BlockSpec((N_PAIR, 16), lambda: (0, 0)),
            pl.BlockSpec((G, 1), lambda: (0, 0)),
            pl.BlockSpec((G, 1), lambda: (0, 0)),
        ),
    )(x, gate_weight, sim, tri)


_NC = 2
_NS = 16
_NW = _NC * _NS


def _sc_scatter_body(x_hbm, dmat_hbm, xs_hbm, idx0_v, idx1_v, rows0_v,
                     rsem, ssem):
    wid = lax.axis_index("s") * _NC + lax.axis_index("c")
    t0 = wid * CHUNK
    c0 = pltpu.async_copy(dmat_hbm.at[wid], idx0_v, rsem)
    c1 = pltpu.async_copy(dmat_hbm.at[wid + _NW], idx1_v, rsem)
    rows_cp = pltpu.async_copy(x_hbm.at[pl.ds(t0, CHUNK)], rows0_v, rsem)
    c0.wait()
    c1.wait()
    rows_cp.wait()
    sc0 = pltpu.async_copy(rows0_v, xs_hbm.at[idx0_v], ssem)
    sc1 = pltpu.async_copy(rows0_v, xs_hbm.at[idx1_v], ssem)
    sc0.wait()
    sc1.wait()


def _sc_scatter(x, d_mat):
    mesh = plsc.VectorSubcoreMesh(core_axis_name="c", subcore_axis_name="s", num_cores=_NC, num_subcores=_NS)
    f = pl.kernel(
        _sc_scatter_body,
        out_type=jax.ShapeDtypeStruct((P_ROWS, D), jnp.float32),
        mesh=mesh,
        scratch_types=[
            pltpu.VMEM((CHUNK,), jnp.int32),
            pltpu.VMEM((CHUNK,), jnp.int32),
            pltpu.VMEM((CHUNK, D), jnp.float32),
            pltpu.SemaphoreType.DMA,
            pltpu.SemaphoreType.DMA,
        ],
    )
    return f(x, d_mat)


def _gemm_body(bmap_ref, bexp_ref, xs_ref, gup_ref, down_ref, y_ref):
    g = pl.program_id(0)

    @pl.when(bmap_ref[g, 0] == g)
    def _():
        xs = xs_ref[...]
        gu = lax.dot_general(xs, gup_ref[0], (((1,), (1,)), ((), ())),
                             preferred_element_type=jnp.float32)
        gate = gu[:, :DFF]
        up = gu[:, DFF:]
        h = gate * jax.nn.sigmoid(gate) * up
        y_ref[...] = lax.dot_general(h, down_ref[0], (((1,), (1,)), ((), ())),
                                     preferred_element_type=jnp.float32)


def _ragged_gemm(bmap, bexp, xs, gate_up_proj, down_proj):
    grid_spec = pltpu.PrefetchScalarGridSpec(
        num_scalar_prefetch=2,
        grid=(G,),
        in_specs=[
            pl.BlockSpec((BT, D), lambda g, bm, be: (bm[g, 0], 0)),
            pl.BlockSpec((1, 2 * DFF, D), lambda g, bm, be: (be[g, 0], 0, 0)),
            pl.BlockSpec((1, D, DFF), lambda g, bm, be: (be[g, 0], 0, 0)),
        ],
        out_specs=pl.BlockSpec((BT, D), lambda g, bm, be: (bm[g, 0], 0)),
    )
    return pl.pallas_call(
        _gemm_body,
        grid_spec=grid_spec,
        out_shape=jax.ShapeDtypeStruct((P_ROWS, D), jnp.float32),
    )(bmap, bexp, xs, gate_up_proj, down_proj)


_TSUB = 16


_N_SUB = CHUNK // _TSUB


def _sc_combine_body(y_hbm, dmat_hbm, ws_hbm, out_hbm,
                     i0all, i1all, w0all, w1all,
                     aa, ab, ba, bb, oa, ob,
                     lsem, gsa, gsb, osa, osb):
    wid = lax.axis_index("s") * _NC + lax.axis_index("c")
    av = (aa, ab)
    bv = (ba, bb)
    ov = (oa, ob)
    gsem = (gsa, gsb)
    osem = (osa, osb)

    row_base = wid * CHUNK
    hoist = (
        pltpu.async_copy(dmat_hbm.at[wid], i0all, lsem),
        pltpu.async_copy(dmat_hbm.at[wid + _NW], i1all, lsem),
        pltpu.async_copy(ws_hbm.at[pl.ds(row_base, CHUNK)], w0all, lsem),
        pltpu.async_copy(ws_hbm.at[pl.ds(N_TOK + row_base, CHUNK)], w1all,
                         lsem),
    )
    for cp in hoist:
        cp.wait()

    def issue(s, bi):
        col = s * _TSUB
        ca = pltpu.async_copy(y_hbm.at[i0all[pl.ds(col, _TSUB)]], av[bi],
                              gsem[bi])
        cb = pltpu.async_copy(y_hbm.at[i1all[pl.ds(col, _TSUB)]], bv[bi],
                              gsem[bi])
        return ca, cb

    pend = issue(0, 0)
    out_pend = [None, None]
    for s in range(_N_SUB):
        bi = s % 2
        col = s * _TSUB
        pend[0].wait()
        pend[1].wait()
        if s + 1 < _N_SUB:
            pend = issue(s + 1, 1 - bi)
        if out_pend[bi] is not None:
            out_pend[bi].wait()

        def row(r, _, bi=bi, col=col):
            s0 = w0all[col + r, :]
            s1 = w1all[col + r, :]

            @plsc.parallel_loop(0, D, step=16, unroll=8)
            def _(jj):
                ov[bi][r, pl.ds(jj, 16)] = (
                    s0 * av[bi][r, pl.ds(jj, 16)]
                    + s1 * bv[bi][r, pl.ds(jj, 16)])

            return 0

        lax.fori_loop(0, _TSUB, row, 0)
        out_pend[bi] = pltpu.async_copy(
            ov[bi], out_hbm.at[pl.ds(row_base + col, _TSUB)], osem[bi])
    for cp in out_pend:
        if cp is not None:
            cp.wait()


def _sc_combine(y, d_mat, w_splat):
    mesh = plsc.VectorSubcoreMesh(core_axis_name="c", subcore_axis_name="s", num_cores=_NC, num_subcores=_NS)
    f = pl.kernel(
        _sc_combine_body,
        out_type=jax.ShapeDtypeStruct((N_TOK, D), jnp.float32),
        mesh=mesh,
        scratch_types=(
            [pltpu.VMEM((CHUNK,), jnp.int32)] * 2
            + [pltpu.VMEM((CHUNK, 16), jnp.float32)] * 2
            + [pltpu.VMEM((_TSUB, D), jnp.float32)] * 6
            + [pltpu.SemaphoreType.DMA] * 5
        ),
    )
    return f(y, d_mat, w_splat)


def kernel(hidden_states, gate_weight, gate_up_proj, down_proj, similarity_matrix):
    B, S, Dm = hidden_states.shape
    x = hidden_states.reshape(-1, Dm)
    tri = jnp.tril(jnp.ones((256, 256), jnp.float32))
    d, w_splat, bmap, bexp = _routing(x, gate_weight, similarity_matrix, tri)
    d_mat = d.reshape(N_CHUNK, CHUNK)
    xs = _sc_scatter(x, d_mat)
    y = _ragged_gemm(bmap, bexp, xs, gate_up_proj, down_proj)
    out = _sc_combine(y, d_mat, w_splat)
    return out.reshape(B, S, Dm)


# speedup vs baseline: 1.6700x; 1.1412x over previous
import jax
import jax.numpy as jnp
from jax.experimental import pallas as pl
from jax.experimental.pallas import tpu as pltpu

N_EXP = 8
D = 1024
DFF = 512
N_TOK = 2048
NEG = -3.0e38


def _argmax_lanes(v, iota_row):
    m = jnp.max(v, axis=-1, keepdims=True)
    return jnp.min(jnp.where(v == m, iota_row, N_EXP), axis=-1, keepdims=True), m


def _routing_body(x_ref, gw_ref, sim_ref, rw_ref):
    x = x_ref[...]
    gw = gw_ref[...]
    logits = jax.lax.dot_general(x, gw, (((1,), (1,)), ((), ())),
                                 preferred_element_type=jnp.float32)
    m = jnp.max(logits, axis=-1, keepdims=True)
    e = jnp.exp(logits - m)
    probs = e / jnp.sum(e, axis=-1, keepdims=True)

    iota_row = jax.lax.broadcasted_iota(jnp.int32, (N_TOK, N_EXP), 1)
    i1, v1 = _argmax_lanes(probs, iota_row)
    oh1 = (iota_row == i1)
    probs2 = jnp.where(oh1, NEG, probs)
    i2, v2 = _argmax_lanes(probs2, iota_row)
    oh2 = (iota_row == i2)
    denom = jnp.maximum(v1 + v2, 1e-12)
    w1 = v1 / denom
    w2 = v2 / denom

    mask_col = jnp.max(oh1.astype(jnp.float32), axis=0, keepdims=True)

    sim = sim_ref[...]
    iota_r8 = jax.lax.broadcasted_iota(jnp.int32, (N_EXP, N_EXP), 1)
    iota_c8 = jax.lax.broadcasted_iota(jnp.int32, (N_EXP, N_EXP), 0)
    eye = (iota_r8 == iota_c8)
    maskb = mask_col > 0.5
    sim_masked = jnp.where(maskb, sim, NEG)
    best_sim = jnp.max(sim_masked, axis=-1, keepdims=True)
    best_j = jnp.min(jnp.where(sim_masked == best_sim, iota_r8, N_EXP),
                     axis=-1, keepdims=True)
    mask_row = jnp.max(jnp.where(eye, jnp.broadcast_to(mask_col, (N_EXP, N_EXP)),
                                 0.0), axis=-1, keepdims=True)
    reroute = (mask_row < 0.5) & (best_sim >= 0.5)
    ident = jax.lax.broadcasted_iota(jnp.int32, (N_EXP, 1), 0)
    emap = jnp.where(reroute, best_j, ident)
    perm = (emap == iota_r8).astype(jnp.float32)

    pre = w1 * oh1.astype(jnp.float32) + w2 * oh2.astype(jnp.float32)
    rw = jax.lax.dot_general(pre, perm, (((1,), (0,)), ((), ())),
                             preferred_element_type=jnp.float32)
    rw_ref[...] = rw


def _ffn_body(x_ref, gup_ref, down_ref, w_ref, out_ref):
    e = pl.program_id(0)
    x = x_ref[...]
    gup = gup_ref[0]
    gu = jax.lax.dot_general(x, gup, (((1,), (1,)), ((), ())),
                             preferred_element_type=jnp.float32)
    gate = gu[:, :DFF]
    up = gu[:, DFF:]
    h = gate * jax.nn.sigmoid(gate) * up
    y = jax.lax.dot_general(h, down_ref[0], (((1,), (1,)), ((), ())),
                            preferred_element_type=jnp.float32)
    lanes = jax.lax.broadcasted_iota(jnp.int32, (N_TOK, N_EXP), 1)
    w_col = jnp.sum(jnp.where(lanes == e, w_ref[...], 0.0), axis=-1,
                    keepdims=True)
    y = y * w_col

    @pl.when(e == 0)
    def _():
        out_ref[...] = y

    @pl.when(e != 0)
    def _():
        out_ref[...] += y


def _routing(x, gate_weight, sim):
    return pl.pallas_call(
        _routing_body,
        out_shape=jax.ShapeDtypeStruct((N_TOK, N_EXP), jnp.float32),
        in_specs=[
            pl.BlockSpec((N_TOK, D), lambda: (0, 0)),
            pl.BlockSpec((N_EXP, D), lambda: (0, 0)),
            pl.BlockSpec((N_EXP, N_EXP), lambda: (0, 0)),
        ],
        out_specs=pl.BlockSpec((N_TOK, N_EXP), lambda: (0, 0)),
    )(x, gate_weight, sim)


def _ffn_dense(x, gate_up_proj, down_proj, rw):
    return pl.pallas_call(
        _ffn_body,
        grid=(N_EXP,),
        out_shape=jax.ShapeDtypeStruct((N_TOK, D), jnp.float32),
        in_specs=[
            pl.BlockSpec((N_TOK, D), lambda e: (0, 0)),
            pl.BlockSpec((1, 2 * DFF, D), lambda e: (e, 0, 0)),
            pl.BlockSpec((1, D, DFF), lambda e: (e, 0, 0)),
            pl.BlockSpec((N_TOK, N_EXP), lambda e: (0, 0)),
        ],
        out_specs=pl.BlockSpec((N_TOK, D), lambda e: (0, 0)),
    )(x, gate_up_proj, down_proj, rw)


def kernel(hidden_states, gate_weight, gate_up_proj, down_proj, similarity_matrix):
    B, S, Dm = hidden_states.shape
    x = hidden_states.reshape(-1, Dm)
    rw = _routing(x, gate_weight, similarity_matrix)
    out = _ffn_dense(x, gate_up_proj, down_proj, rw)
    return out.reshape(B, S, Dm)
